# Initial kernel scaffold; baseline (speedup 1.0000x reference)
import jax
import jax.numpy as jnp
from jax.experimental import pallas as pl


def kernel(entity_emb, user_emb, edge_index, edge_type, interact_mat, weight):
    raise NotImplementedError("write your pallas kernel here")


# speedup vs baseline: 4.9522x; 4.9522x over previous
import functools

import jax
import jax.numpy as jnp
from jax import lax
from jax.experimental import pallas as pl
from jax.experimental.pallas import tpu as pltpu
from jax.experimental.pallas import tpu_sc as plsc

NC, NS, L = 2, 16, 16
NW = NC * NS
CH = 128
SLICE = 3136
NPAD = SLICE * NS
NVS = SLICE // L
MCH = 2000
ZR = 112


_GDN = lax.GatherDimensionNumbers(
    offset_dims=(), collapsed_slice_dims=(0,), start_index_map=(0,))


def _vgather(x, idx):
    return lax.gather(x, idx[:, None], _GDN, (1,),
                      mode=lax.GatherScatterMode.PROMISE_IN_BOUNDS)


def _seg_update(m_ref, lane_t, idx, val):
    iota = lax.iota(jnp.int32, L)
    fiota = iota.astype(jnp.float32)
    plsc.store_scatter(lane_t, [idx], fiota, mask=idx >= 0)
    got = plsc.load_gather(lane_t, [idx])
    cur = plsc.load_gather(m_ref, [idx])
    leader = got == fiota
    has_dup = jnp.any(jnp.logical_not(leader))

    @pl.when(jnp.logical_not(has_dup))
    def _():
        plsc.store_scatter(m_ref, [idx], val, mask=val > cur)

    @pl.when(has_dup)
    def _():
        vmax = val
        for d in range(1, L):
            src = (iota + d) & (L - 1)
            oi = _vgather(idx, src)
            ov = _vgather(val, src)
            vmax = jnp.where(oi == idx, jnp.maximum(vmax, ov), vmax)
        plsc.store_scatter(m_ref, [idx], vmax, mask=leader & (vmax > cur))


def _merge_slice(part_hbm, out_ref, core, sid, acc_b, stg_b, combine):
    off = sid * SLICE
    row0 = core * NS * NPAD
    pltpu.sync_copy(part_hbm.at[pl.ds(row0 + off, SLICE)], acc_b)

    def one_row(j, _):
        pltpu.sync_copy(part_hbm.at[pl.ds(row0 + j * NPAD + off, SLICE)], stg_b)

        def one_vreg(q, _):
            sl = pl.ds(q * L, L)
            acc_b[sl] = combine(acc_b[sl], stg_b[sl])
            return 0

        return lax.fori_loop(0, NVS, one_vreg, 0)

    lax.fori_loop(1, NS, one_row, 0)
    pltpu.sync_copy(acc_b, out_ref.at[pl.ds(core * NPAD + off, SLICE)])


def _load_merged(src_ref, dst_ref, stg_b, combine, n):
    pltpu.sync_copy(src_ref.at[pl.ds(0, n)], dst_ref.at[pl.ds(0, n)])

    def one_chunk(p, _):
        pltpu.sync_copy(src_ref.at[pl.ds(NPAD + p * MCH, MCH)], stg_b)

        def one_vreg(q, _):
            sl = pl.ds(p * MCH + q * L, L)
            dst_ref[sl] = combine(dst_ref[sl], stg_b[pl.ds(q * L, L)])
            return 0

        return lax.fori_loop(0, MCH // L, one_vreg, 0)

    lax.fori_loop(0, n // MCH, one_chunk, 0)


def _fill(ref, n, value):
    vec = jnp.full((L,), value, ref.dtype)

    def one(i, _):
        ref[pl.ds(i * L, L)] = vec
        return 0

    lax.fori_loop(0, n // L, one, 0)


def _prep_body(ent_ref, w_ref, lo_ref, hi_ref, nt_ref):
    ent = ent_ref[...]
    w = w_ref[...]
    prod = ent[:, None, :] * w[None, :, :]
    rb = ent.shape[0]
    lo_ref[...] = prod[:, :, :32].reshape(rb * 16, 32)
    hi_ref[...] = prod[:, :, 32:].reshape(rb * 16, 32)
    nt_ref[...] = jnp.sqrt(jnp.dot(
        ent * ent, (w * w).T,
        preferred_element_type=jnp.float32,
        precision=lax.Precision.HIGHEST))


def _tc_prep(entity_emb, weight):
    n_ent, d = entity_emb.shape
    n_rel = weight.shape[0]
    rb = 400
    grid = n_ent // rb
    return pl.pallas_call(
        _prep_body,
        grid=(grid,),
        in_specs=[
            pl.BlockSpec((rb, d), lambda i: (i, 0)),
            pl.BlockSpec((n_rel, d), lambda i: (0, 0)),
        ],
        out_specs=[
            pl.BlockSpec((rb * n_rel, 32), lambda i: (i, 0)),
            pl.BlockSpec((rb * n_rel, 32), lambda i: (i, 0)),
            pl.BlockSpec((rb, n_rel), lambda i: (i, 0)),
        ],
        out_shape=[
            jax.ShapeDtypeStruct((n_ent * n_rel, 32), jnp.float32),
            jax.ShapeDtypeStruct((n_ent * n_rel, 32), jnp.float32),
            jax.ShapeDtypeStruct((n_ent, n_rel), jnp.float32),
        ],
    )(entity_emb, weight)


def _make_sc1(n_ent, n_edge):
    ncht = n_edge // CH
    base_chunks = ncht // NW
    extra = ncht % NW
    mesh = plsc.VectorSubcoreMesh(core_axis_name="c", subcore_axis_name="s")

    def body(nt_hbm, head_hbm, tail_hbm, et_hbm,
             a_hbm, g_hbm, msc_hbm, mpart_hbm,
             m_t, lane_t, h_b, t_b, r_b, nth_b, ntt_b, a_b, g_b,
             acc_b, stg_b, sem1, sem2):
        c = lax.axis_index("c")
        s = lax.axis_index("s")
        wid = c * NS + s
        _fill(m_t, n_ent, -1.0)
        nch = base_chunks + (wid < extra).astype(jnp.int32)

        def one_chunk(j, _):
            base = (wid + NW * j) * CH
            pltpu.sync_copy(head_hbm.at[pl.ds(base, CH)], h_b)
            pltpu.sync_copy(tail_hbm.at[pl.ds(base, CH)], t_b)
            pltpu.sync_copy(et_hbm.at[pl.ds(base, CH)], r_b)
            for k in range(CH // L):
                sl = pl.ds(k * L, L)
                ridx = (r_b[sl] - 1) & 15
                r_b[sl] = h_b[sl] * 16 + ridx
                g_b[sl] = t_b[sl] * 16 + ridx
            d1 = pltpu.async_copy(nt_hbm.at[r_b], nth_b, sem1)
            d2 = pltpu.async_copy(nt_hbm.at[g_b], ntt_b, sem2)
            d1.wait()
            d2.wait()
            for k in range(CH // L):
                sl = pl.ds(k * L, L)
                p = nth_b[sl] * ntt_b[sl]
                a = p * p
                a_b[sl] = a
                _seg_update(m_t, lane_t, h_b[sl], a)
            pltpu.sync_copy(a_b, a_hbm.at[pl.ds(base, CH)])
            pltpu.sync_copy(g_b, g_hbm.at[pl.ds(base, CH)])
            return 0

        lax.fori_loop(0, nch, one_chunk, 0)

        pltpu.sync_copy(m_t, mpart_hbm.at[pl.ds(wid * NPAD, n_ent)])
        plsc.subcore_barrier()
        _merge_slice(mpart_hbm, msc_hbm, c, s, acc_b, stg_b, jnp.maximum)

    return pl.kernel(
        body,
        out_type=(
            jax.ShapeDtypeStruct((n_edge,), jnp.float32),
            jax.ShapeDtypeStruct((n_edge,), jnp.int32),
            jax.ShapeDtypeStruct((NC * NPAD,), jnp.float32),
            jax.ShapeDtypeStruct((NW * NPAD,), jnp.float32),
        ),
        mesh=mesh,
        compiler_params=pltpu.CompilerParams(
            needs_layout_passes=False, use_tc_tiling_on_sc=False),
        scratch_types=[
            pltpu.VMEM((n_ent,), jnp.float32),
            pltpu.VMEM((n_ent,), jnp.float32),
            pltpu.VMEM((CH,), jnp.int32),
            pltpu.VMEM((CH,), jnp.int32),
            pltpu.VMEM((CH,), jnp.int32),
            pltpu.VMEM((CH,), jnp.float32),
            pltpu.VMEM((CH,), jnp.float32),
            pltpu.VMEM((CH,), jnp.float32),
            pltpu.VMEM((CH,), jnp.int32),
            pltpu.VMEM((SLICE,), jnp.float32),
            pltpu.VMEM((SLICE,), jnp.float32),
            pltpu.SemaphoreType.DMA,
            pltpu.SemaphoreType.DMA,
        ],
    )


def _make_sc2(n_ent, n_edge):
    ncht = n_edge // CH
    base_chunks = ncht // NW
    extra = ncht % NW
    mesh = plsc.VectorSubcoreMesh(core_axis_name="c", subcore_axis_name="s")

    def body(head_hbm, a_hbm, msc_hbm,
             e_hbm, dsc_hbm, dpart_hbm,
             m_t, d_t, h_b, a_b, e_b, mstg_b, acc_b, stg_b):
        c = lax.axis_index("c")
        s = lax.axis_index("s")
        wid = c * NS + s
        _load_merged(msc_hbm, m_t, mstg_b, jnp.maximum, n_ent)
        _fill(d_t, n_ent, 0.0)
        nch = base_chunks + (wid < extra).astype(jnp.int32)

        def one_chunk(j, _):
            base = (wid + NW * j) * CH
            pltpu.sync_copy(head_hbm.at[pl.ds(base, CH)], h_b)
            pltpu.sync_copy(a_hbm.at[pl.ds(base, CH)], a_b)
            for k in range(CH // L):
                sl = pl.ds(k * L, L)
                h = h_b[sl]
                mv = plsc.load_gather(m_t, [h])
                e = jnp.exp(a_b[sl] - mv)
                e_b[sl] = e
                plsc.addupdate_scatter(d_t, [h], e)
            pltpu.sync_copy(e_b, e_hbm.at[pl.ds(base, CH)])
            return 0

        lax.fori_loop(0, nch, one_chunk, 0)

        pltpu.sync_copy(d_t, dpart_hbm.at[pl.ds(wid * NPAD, n_ent)])
        plsc.subcore_barrier()
        _merge_slice(dpart_hbm, dsc_hbm, c, s, acc_b, stg_b, jnp.add)

    return pl.kernel(
        body,
        out_type=(
            jax.ShapeDtypeStruct((n_edge,), jnp.float32),
            jax.ShapeDtypeStruct((NC * NPAD,), jnp.float32),
            jax.ShapeDtypeStruct((NW * NPAD,), jnp.float32),
        ),
        mesh=mesh,
        compiler_params=pltpu.CompilerParams(
            needs_layout_passes=False, use_tc_tiling_on_sc=False),
        scratch_types=[
            pltpu.VMEM((n_ent,), jnp.float32),
            pltpu.VMEM((n_ent,), jnp.float32),
            pltpu.VMEM((CH,), jnp.int32),
            pltpu.VMEM((CH,), jnp.float32),
            pltpu.VMEM((CH,), jnp.float32),
            pltpu.VMEM((MCH,), jnp.float32),
            pltpu.VMEM((SLICE,), jnp.float32),
            pltpu.VMEM((SLICE,), jnp.float32),
        ],
    )


def _make_sc2b(n_ent, n_edge):
    ncht = n_edge // CH
    base_chunks = ncht // NW
    extra = ncht % NW
    mesh = plsc.VectorSubcoreMesh(core_axis_name="c", subcore_axis_name="s")

    def body(head_hbm, e_hbm, dsc_hbm, s_hbm,
             d_t, h_b, e_b, s_b, mstg_b):
        c = lax.axis_index("c")
        s = lax.axis_index("s")
        wid = c * NS + s
        _load_merged(dsc_hbm, d_t, mstg_b, jnp.add, n_ent)
        nch = base_chunks + (wid < extra).astype(jnp.int32)

        def one_chunk(j, _):
            base = (wid + NW * j) * CH
            pltpu.sync_copy(head_hbm.at[pl.ds(base, CH)], h_b)
            pltpu.sync_copy(e_hbm.at[pl.ds(base, CH)], e_b)
            for k in range(CH // L):
                sl = pl.ds(k * L, L)
                dv = plsc.load_gather(d_t, [h_b[sl]])
                s_b[sl] = e_b[sl] / dv
            pltpu.sync_copy(s_b, s_hbm.at[pl.ds(base, CH)])
            return 0

        lax.fori_loop(0, nch, one_chunk, 0)

    return pl.kernel(
        body,
        out_type=(jax.ShapeDtypeStruct((n_edge,), jnp.float32),),
        mesh=mesh,
        compiler_params=pltpu.CompilerParams(
            needs_layout_passes=False, use_tc_tiling_on_sc=False),
        scratch_types=[
            pltpu.VMEM((n_ent,), jnp.float32),
            pltpu.VMEM((CH,), jnp.int32),
            pltpu.VMEM((CH,), jnp.float32),
            pltpu.VMEM((CH,), jnp.float32),
            pltpu.VMEM((MCH,), jnp.float32),
        ],
    )


def _make_sc3(n_ent, n_edge):
    ncht = n_edge // CH
    base_chunks = ncht // NS
    extra = ncht % NS
    mesh = plsc.VectorSubcoreMesh(core_axis_name="c", subcore_axis_name="s")

    def body(lo_hbm, hi_hbm, head_hbm, g_hbm, s_hbm,
             out_lo, out_hi,
             h_b, g_b, s_b, rows, z_b, st_b, agg, sem):
        c = lax.axis_index("c")
        s = lax.axis_index("s")
        for i in range(ZR):
            for j in range(2):
                z_b[i, pl.ds(j * L, L)] = jnp.zeros((L,), jnp.float32)
        off = s * SLICE

        def zloop(q, _):
            pltpu.sync_copy(z_b, agg.at[pl.ds(off + q * ZR, ZR)])
            return 0

        lax.fori_loop(0, SLICE // ZR, zloop, 0)
        plsc.subcore_barrier()

        nch = base_chunks + (s < extra).astype(jnp.int32)

        def one_chunk(j, _):
            base = (s + NS * j) * CH
            pltpu.sync_copy(head_hbm.at[pl.ds(base, CH)], h_b)
            pltpu.sync_copy(g_hbm.at[pl.ds(base, CH)], g_b)
            pltpu.sync_copy(s_hbm.at[pl.ds(base, CH)], s_b)

            @pl.when(c == 0)
            def _():
                pltpu.async_copy(lo_hbm.at[g_b], rows, sem).wait()

            @pl.when(c == 1)
            def _():
                pltpu.async_copy(hi_hbm.at[g_b], rows, sem).wait()

            for k in range(CH // L):
                sv = s_b[pl.ds(k * L, L)]
                for i in range(L):
                    row = k * L + i
                    sp = sv[i]
                    rows[row, 0:L] = rows[row, 0:L] * sp
                    rows[row, L:2 * L] = rows[row, L:2 * L] * sp
            pltpu.sync_copy(rows, agg.at[h_b], add=True)
            return 0

        lax.fori_loop(0, nch, one_chunk, 0)
        plsc.subcore_barrier()

        def drain(q, _):
            pltpu.sync_copy(agg.at[pl.ds(off + q * ZR, ZR)], st_b)

            @pl.when(c == 0)
            def _():
                pltpu.sync_copy(st_b, out_lo.at[pl.ds(off + q * ZR, ZR)])

            @pl.when(c == 1)
            def _():
                pltpu.sync_copy(st_b, out_hi.at[pl.ds(off + q * ZR, ZR)])

            return 0

        lax.fori_loop(0, SLICE // ZR, drain, 0)

    return pl.kernel(
        body,
        out_type=(
            jax.ShapeDtypeStruct((NPAD, 32), jnp.float32),
            jax.ShapeDtypeStruct((NPAD, 32), jnp.float32),
        ),
        mesh=mesh,
        compiler_params=pltpu.CompilerParams(
            needs_layout_passes=False, use_tc_tiling_on_sc=False),
        scratch_types=[
            pltpu.VMEM((CH,), jnp.int32),
            pltpu.VMEM((CH,), jnp.int32),
            pltpu.VMEM((CH,), jnp.float32),
            pltpu.VMEM((CH, 32), jnp.float32),
            pltpu.VMEM((ZR, 32), jnp.float32),
            pltpu.VMEM((ZR, 32), jnp.float32),
            pltpu.VMEM_SHARED((NPAD, 32), jnp.float32),
            pltpu.SemaphoreType.DMA,
        ],
    )


def _make_final_body(n_ent, eb, grid):
    tail = n_ent - (grid - 1) * eb

    def body(im_ref, ent_ref, u_ref, lo_ref, hi_ref, eagg_ref, uagg_ref):
        i = pl.program_id(0)
        im = im_ref[...]
        ent = ent_ref[...]

        @pl.when(i == grid - 1)
        def _():
            cols = lax.broadcasted_iota(jnp.int32, im.shape, 1)
            rows = lax.broadcasted_iota(jnp.int32, ent.shape, 0)
            im_ref[...] = jnp.where(cols < tail, im, 0.0)
            ent_ref[...] = jnp.where(rows < tail, ent, 0.0)

        imz = im_ref[...]
        base = lax.dot_general(imz, u_ref[...], (((0,), (0,)), ((), ())),
                               preferred_element_type=jnp.float32)
        eagg_ref[:, 0:32] = lo_ref[...] + base[:, 0:32]
        eagg_ref[:, 32:64] = hi_ref[...] + base[:, 32:64]

        @pl.when(i == 0)
        def _():
            uagg_ref[...] = jnp.zeros_like(uagg_ref)

        uagg_ref[...] += jnp.dot(imz, ent_ref[...],
                                 preferred_element_type=jnp.float32)

    return body


def _tc_final(interact_mat, entity_emb, user_emb, sc_lo, sc_hi):
    n_usr, n_ent = interact_mat.shape
    d = entity_emb.shape[1]
    eb = 2048
    grid = (n_ent + eb - 1) // eb
    return pl.pallas_call(
        _make_final_body(n_ent, eb, grid),
        grid=(grid,),
        in_specs=[
            pl.BlockSpec((n_usr, eb), lambda i: (0, i)),
            pl.BlockSpec((eb, d), lambda i: (i, 0)),
            pl.BlockSpec((n_usr, d), lambda i: (0, 0)),
            pl.BlockSpec((eb, 32), lambda i: (i, 0)),
            pl.BlockSpec((eb, 32), lambda i: (i, 0)),
        ],
        out_specs=[
            pl.BlockSpec((eb, d), lambda i: (i, 0)),
            pl.BlockSpec((n_usr, d), lambda i: (0, 0)),
        ],
        out_shape=[
            jax.ShapeDtypeStruct((n_ent, d), jnp.float32),
            jax.ShapeDtypeStruct((n_usr, d), jnp.float32),
        ],
        compiler_params=pltpu.CompilerParams(
            dimension_semantics=("arbitrary",)),
    )(interact_mat, entity_emb, user_emb, sc_lo, sc_hi)


def kernel(entity_emb, user_emb, edge_index, edge_type, interact_mat, weight):
    n_ent = entity_emb.shape[0]
    n_edge = edge_index.shape[1]
    head = edge_index[0]
    tail = edge_index[1]

    lo, hi, nt = _tc_prep(entity_emb, weight)
    a_e, g_idx, m_sc, _ = _make_sc1(n_ent, n_edge)(
        nt.reshape(-1), head, tail, edge_type)
    e_e, d_sc, _ = _make_sc2(n_ent, n_edge)(head, a_e, m_sc)
    (s_e,) = _make_sc2b(n_ent, n_edge)(head, e_e, d_sc)
    sc_lo, sc_hi = _make_sc3(n_ent, n_edge)(lo, hi, head, g_idx, s_e)
    entity_agg, user_agg = _tc_final(
        interact_mat, entity_emb, user_emb,
        sc_lo[:n_ent], sc_hi[:n_ent])
    return (entity_agg, user_agg)


# speedup vs baseline: 8.5039x; 1.7172x over previous
import jax
import jax.numpy as jnp
from jax import lax
from jax.experimental import pallas as pl
from jax.experimental.pallas import tpu as pltpu
from jax.experimental.pallas import tpu_sc as plsc

NC, NS, L = 2, 16, 16
NW = NC * NS
CH = 128
KV = CH // L
SLICE = 3136
NPAD = SLICE * NS
NVS = SLICE // L
MCH = 2000
ZR = 112

_SC_PARAMS = pltpu.CompilerParams(
    needs_layout_passes=False, use_tc_tiling_on_sc=False)

_GDN = lax.GatherDimensionNumbers(
    offset_dims=(), collapsed_slice_dims=(0,), start_index_map=(0,))


def _vgather(x, idx):
    return lax.gather(x, idx[:, None], _GDN, (1,),
                      mode=lax.GatherScatterMode.PROMISE_IN_BOUNDS)


def _seg_update(m_ref, lane_t, idx, val):
    iota = lax.iota(jnp.int32, L)
    fiota = iota.astype(jnp.float32)
    plsc.store_scatter(lane_t, [idx], fiota, mask=idx >= 0)
    got = plsc.load_gather(lane_t, [idx])
    cur = plsc.load_gather(m_ref, [idx])
    leader = got == fiota
    has_dup = jnp.any(jnp.logical_not(leader))

    @pl.when(jnp.logical_not(has_dup))
    def _():
        plsc.store_scatter(m_ref, [idx], val, mask=val > cur)

    @pl.when(has_dup)
    def _():
        vmax = val
        for d in range(1, L):
            src = (iota + d) & (L - 1)
            oi = _vgather(idx, src)
            ov = _vgather(val, src)
            vmax = jnp.where(oi == idx, jnp.maximum(vmax, ov), vmax)
        plsc.store_scatter(m_ref, [idx], vmax, mask=leader & (vmax > cur))


def _merge_slice(part_hbm, out_ref, core, sid, acc_b, stg_b, combine):
    off = sid * SLICE
    row0 = core * NS * NPAD
    pltpu.sync_copy(part_hbm.at[pl.ds(row0 + off, SLICE)], acc_b)

    def one_row(j, _):
        pltpu.sync_copy(part_hbm.at[pl.ds(row0 + j * NPAD + off, SLICE)], stg_b)

        def one_vreg(q, _):
            sl = pl.ds(q * L, L)
            acc_b[sl] = combine(acc_b[sl], stg_b[sl])
            return 0

        return lax.fori_loop(0, NVS, one_vreg, 0)

    lax.fori_loop(1, NS, one_row, 0)
    pltpu.sync_copy(acc_b, out_ref.at[pl.ds(core * NPAD + off, SLICE)])


def _load_merged(src_ref, dst_ref, stg_b, combine, n):
    pltpu.sync_copy(src_ref.at[pl.ds(0, n)], dst_ref.at[pl.ds(0, n)])

    def one_chunk(p, _):
        pltpu.sync_copy(src_ref.at[pl.ds(NPAD + p * MCH, MCH)], stg_b)

        def one_vreg(q, _):
            sl = pl.ds(p * MCH + q * L, L)
            dst_ref[sl] = combine(dst_ref[sl], stg_b[pl.ds(q * L, L)])
            return 0

        return lax.fori_loop(0, MCH // L, one_vreg, 0)

    lax.fori_loop(0, n // MCH, one_chunk, 0)


def _fill(ref, n, value):
    vec = jnp.full((L,), value, ref.dtype)

    def one(i, _):
        ref[pl.ds(i * L, L)] = vec
        return 0

    lax.fori_loop(0, n // L, one, 0)


def _prep_body(ent_ref, w_ref, lo_ref, hi_ref, nt_ref):
    ent = ent_ref[...]
    w = w_ref[...]
    prod = ent[:, None, :] * w[None, :, :]
    rb = ent.shape[0]
    lo_ref[...] = prod[:, :, :32].reshape(rb * 16, 32)
    hi_ref[...] = prod[:, :, 32:].reshape(rb * 16, 32)
    nt_ref[...] = jnp.sqrt(jnp.dot(
        ent * ent, (w * w).T,
        preferred_element_type=jnp.float32,
        precision=lax.Precision.HIGHEST))


def _tc_prep(entity_emb, weight):
    n_ent, d = entity_emb.shape
    n_rel = weight.shape[0]
    rb = 400
    grid = n_ent // rb
    return pl.pallas_call(
        _prep_body,
        grid=(grid,),
        in_specs=[
            pl.BlockSpec((rb, d), lambda i: (i, 0)),
            pl.BlockSpec((n_rel, d), lambda i: (0, 0)),
        ],
        out_specs=[
            pl.BlockSpec((rb * n_rel, 32), lambda i: (i, 0)),
            pl.BlockSpec((rb * n_rel, 32), lambda i: (i, 0)),
            pl.BlockSpec((rb, n_rel), lambda i: (i, 0)),
        ],
        out_shape=[
            jax.ShapeDtypeStruct((n_ent * n_rel, 32), jnp.float32),
            jax.ShapeDtypeStruct((n_ent * n_rel, 32), jnp.float32),
            jax.ShapeDtypeStruct((n_ent, n_rel), jnp.float32),
        ],
    )(entity_emb, weight)


def _make_sc1(n_ent, n_edge):
    ncht = n_edge // CH
    nch_u = -(-ncht // NW)
    nch_u += nch_u % 2
    np_ = nch_u // 2
    mesh = plsc.VectorSubcoreMesh(core_axis_name="c", subcore_axis_name="s")

    def body(nt_hbm, head_hbm, tail_hbm, et_hbm,
             a_hbm, g_hbm, msc_hbm, mpart_hbm,
             m_t, lane_t,
             h0, t0, r0, gh0, gt0, nh0, ntl0, a0,
             h1, t1, r1, gh1, gt1, nh1, ntl1, a1,
             acc_b, stg_b,
             sl0, sl1, sg0, sg1, ss0, ss1):
        c = lax.axis_index("c")
        s = lax.axis_index("s")
        wid = c * NS + s
        _fill(m_t, n_ent, -1.0)
        hb = (h0, h1)
        tb = (t0, t1)
        rb = (r0, r1)
        ghb = (gh0, gh1)
        gtb = (gt0, gt1)
        nhb = (nh0, nh1)
        ntlb = (ntl0, ntl1)
        ab = (a0, a1)
        slin = (sl0, sl1)
        sgat = (sg0, sg1)
        sst = (ss0, ss1)

        def cid_of(j):
            raw = wid + NW * j
            real = raw < ncht
            return jnp.minimum(raw, ncht - 1), real

        def issue_lin(q, j):
            cid, _ = cid_of(j)
            base = cid * CH
            pltpu.async_copy(head_hbm.at[pl.ds(base, CH)], hb[q], slin[q])
            pltpu.async_copy(tail_hbm.at[pl.ds(base, CH)], tb[q], slin[q])
            pltpu.async_copy(et_hbm.at[pl.ds(base, CH)], rb[q], slin[q])

        def wait_lin(q):
            pltpu.make_async_copy(head_hbm.at[pl.ds(0, CH)], hb[q], slin[q]).wait()
            pltpu.make_async_copy(tail_hbm.at[pl.ds(0, CH)], tb[q], slin[q]).wait()
            pltpu.make_async_copy(et_hbm.at[pl.ds(0, CH)], rb[q], slin[q]).wait()

        def drain_store(q):
            pltpu.make_async_copy(ab[q], a_hbm.at[pl.ds(0, CH)], sst[q]).wait()
            pltpu.make_async_copy(gtb[q], g_hbm.at[pl.ds(0, CH)], sst[q]).wait()

        def idx_and_gather(q):
            for k in range(KV):
                sl = pl.ds(k * L, L)
                ridx = (rb[q][sl] - 1) & 15
                ghb[q][sl] = hb[q][sl] * 16 + ridx
                gtb[q][sl] = tb[q][sl] * 16 + ridx
            pltpu.async_copy(nt_hbm.at[ghb[q]], nhb[q], sgat[q])
            pltpu.async_copy(nt_hbm.at[gtb[q]], ntlb[q], sgat[q])

        def wait_gather(q):
            pltpu.make_async_copy(nt_hbm.at[pl.ds(0, CH)], nhb[q], sgat[q]).wait()
            pltpu.make_async_copy(nt_hbm.at[pl.ds(0, CH)], ntlb[q], sgat[q]).wait()

        def main(q, j):
            cid, real = cid_of(j)
            realf = real.astype(jnp.float32)
            for k in range(KV):
                sl = pl.ds(k * L, L)
                p = nhb[q][sl] * ntlb[q][sl]
                a = p * p * realf - (1.0 - realf)
                ab[q][sl] = a
                _seg_update(m_t, lane_t, hb[q][sl], a)
            base = jnp.where(real, cid * CH, n_edge)
            pltpu.async_copy(ab[q], a_hbm.at[pl.ds(base, CH)], sst[q])
            pltpu.async_copy(gtb[q], g_hbm.at[pl.ds(base, CH)], sst[q])

        for q in (0, 1):
            pltpu.async_copy(ab[q], a_hbm.at[pl.ds(n_edge, CH)], sst[q])
            pltpu.async_copy(gtb[q], g_hbm.at[pl.ds(n_edge, CH)], sst[q])
        issue_lin(0, 0)
        issue_lin(1, 1)

        def pair(jj, _):
            j0 = 2 * jj
            drain_store(0)
            wait_lin(0)
            idx_and_gather(0)
            drain_store(1)
            wait_lin(1)
            idx_and_gather(1)
            wait_gather(0)
            main(0, j0)
            issue_lin(0, j0 + 2)
            wait_gather(1)
            main(1, j0 + 3 - 2)
            issue_lin(1, j0 + 3)
            return 0

        lax.fori_loop(0, np_, pair, 0)
        for q in (0, 1):
            drain_store(q)
            wait_lin(q)

        pltpu.sync_copy(m_t, mpart_hbm.at[pl.ds(wid * NPAD, n_ent)])
        plsc.subcore_barrier()
        _merge_slice(mpart_hbm, msc_hbm, c, s, acc_b, stg_b, jnp.maximum)

    cb_i = pltpu.VMEM((CH,), jnp.int32)
    cb_f = pltpu.VMEM((CH,), jnp.float32)
    return pl.kernel(
        body,
        out_type=(
            jax.ShapeDtypeStruct((n_edge + CH,), jnp.float32),
            jax.ShapeDtypeStruct((n_edge + CH,), jnp.int32),
            jax.ShapeDtypeStruct((NC * NPAD,), jnp.float32),
            jax.ShapeDtypeStruct((NW * NPAD,), jnp.float32),
        ),
        mesh=mesh,
        compiler_params=_SC_PARAMS,
        scratch_types=[
            pltpu.VMEM((n_ent,), jnp.float32),
            pltpu.VMEM((n_ent,), jnp.float32),
            cb_i, cb_i, cb_i, cb_i, cb_i, cb_f, cb_f, cb_f,
            cb_i, cb_i, cb_i, cb_i, cb_i, cb_f, cb_f, cb_f,
            pltpu.VMEM((SLICE,), jnp.float32),
            pltpu.VMEM((SLICE,), jnp.float32),
            pltpu.SemaphoreType.DMA, pltpu.SemaphoreType.DMA,
            pltpu.SemaphoreType.DMA, pltpu.SemaphoreType.DMA,
            pltpu.SemaphoreType.DMA, pltpu.SemaphoreType.DMA,
        ],
    )


def _make_sc2(n_ent, n_edge):
    ncht = n_edge // CH
    nch_u = -(-ncht // NW)
    nch_u += nch_u % 2
    np_ = nch_u // 2
    mesh = plsc.VectorSubcoreMesh(core_axis_name="c", subcore_axis_name="s")

    def body(head_hbm, a_hbm, msc_hbm,
             e_hbm, dsc_hbm, dpart_hbm,
             m_t, d_t,
             h0, a0, e0, h1, a1, e1,
             mstg_b, acc_b, stg_b,
             sl0, sl1, ss0, ss1):
        c = lax.axis_index("c")
        s = lax.axis_index("s")
        wid = c * NS + s
        _load_merged(msc_hbm, m_t, mstg_b, jnp.maximum, n_ent)
        _fill(d_t, n_ent, 0.0)
        hb = (h0, h1)
        ab = (a0, a1)
        eb = (e0, e1)
        slin = (sl0, sl1)
        sst = (ss0, ss1)

        def cid_of(j):
            raw = wid + NW * j
            real = raw < ncht
            return jnp.minimum(raw, ncht - 1), real

        def issue_lin(q, j):
            cid, _ = cid_of(j)
            base = cid * CH
            pltpu.async_copy(head_hbm.at[pl.ds(base, CH)], hb[q], slin[q])
            pltpu.async_copy(a_hbm.at[pl.ds(base, CH)], ab[q], slin[q])

        def step(q, j):
            pltpu.make_async_copy(eb[q], e_hbm.at[pl.ds(0, CH)], sst[q]).wait()
            pltpu.make_async_copy(head_hbm.at[pl.ds(0, CH)], hb[q], slin[q]).wait()
            pltpu.make_async_copy(a_hbm.at[pl.ds(0, CH)], ab[q], slin[q]).wait()
            cid, real = cid_of(j)
            realf = real.astype(jnp.float32)
            for k in range(KV):
                sl = pl.ds(k * L, L)
                h = hb[q][sl]
                mv = plsc.load_gather(m_t, [h])
                e = jnp.exp(ab[q][sl] - mv) * realf
                eb[q][sl] = e
                plsc.addupdate_scatter(d_t, [h], e)
            base = jnp.where(real, cid * CH, n_edge)
            pltpu.async_copy(eb[q], e_hbm.at[pl.ds(base, CH)], sst[q])
            issue_lin(q, j + 2)

        for q in (0, 1):
            pltpu.async_copy(eb[q], e_hbm.at[pl.ds(n_edge, CH)], sst[q])
        issue_lin(0, 0)
        issue_lin(1, 1)

        def pair(jj, _):
            step(0, 2 * jj)
            step(1, 2 * jj + 1)
            return 0

        lax.fori_loop(0, np_, pair, 0)
        for q in (0, 1):
            pltpu.make_async_copy(eb[q], e_hbm.at[pl.ds(0, CH)], sst[q]).wait()
            pltpu.make_async_copy(head_hbm.at[pl.ds(0, CH)], hb[q], slin[q]).wait()
            pltpu.make_async_copy(a_hbm.at[pl.ds(0, CH)], ab[q], slin[q]).wait()

        pltpu.sync_copy(d_t, dpart_hbm.at[pl.ds(wid * NPAD, n_ent)])
        plsc.subcore_barrier()
        _merge_slice(dpart_hbm, dsc_hbm, c, s, acc_b, stg_b, jnp.add)

    cb_i = pltpu.VMEM((CH,), jnp.int32)
    cb_f = pltpu.VMEM((CH,), jnp.float32)
    return pl.kernel(
        body,
        out_type=(
            jax.ShapeDtypeStruct((n_edge + CH,), jnp.float32),
            jax.ShapeDtypeStruct((NC * NPAD,), jnp.float32),
            jax.ShapeDtypeStruct((NW * NPAD,), jnp.float32),
        ),
        mesh=mesh,
        compiler_params=_SC_PARAMS,
        scratch_types=[
            pltpu.VMEM((n_ent,), jnp.float32),
            pltpu.VMEM((n_ent,), jnp.float32),
            cb_i, cb_f, cb_f, cb_i, cb_f, cb_f,
            pltpu.VMEM((MCH,), jnp.float32),
            pltpu.VMEM((SLICE,), jnp.float32),
            pltpu.VMEM((SLICE,), jnp.float32),
            pltpu.SemaphoreType.DMA, pltpu.SemaphoreType.DMA,
            pltpu.SemaphoreType.DMA, pltpu.SemaphoreType.DMA,
        ],
    )


def _make_sc3(n_ent, n_edge):
    ncht = n_edge // CH
    nch_u = -(-ncht // NS)
    nch_u += nch_u % 2
    np_ = nch_u // 2
    mesh = plsc.VectorSubcoreMesh(core_axis_name="c", subcore_axis_name="s")

    def body(lo_hbm, hi_hbm, head_hbm, g_hbm, e_hbm,
             out_lo, out_hi,
             h0, g0, e0, rows0, h1, g1, e1, rows1,
             z_b, st_b, agg,
             sl0, sl1, sr0, sr1):
        c = lax.axis_index("c")
        s = lax.axis_index("s")
        hb = (h0, h1)
        gb = (g0, g1)
        eb = (e0, e1)
        rows = (rows0, rows1)
        slin = (sl0, sl1)
        srow = (sr0, sr1)

        for i in range(ZR):
            for j in range(2):
                z_b[i, pl.ds(j * L, L)] = jnp.zeros((L,), jnp.float32)
        off = s * SLICE

        def zloop(q, _):
            pltpu.sync_copy(z_b, agg.at[pl.ds(off + q * ZR, ZR)])
            return 0

        lax.fori_loop(0, SLICE // ZR, zloop, 0)
        plsc.subcore_barrier()

        def cid_of(j):
            raw = s + NS * j
            real = raw < ncht
            return jnp.minimum(raw, ncht - 1), real

        def issue_lin(q, j):
            cid, _ = cid_of(j)
            base = cid * CH
            pltpu.async_copy(head_hbm.at[pl.ds(base, CH)], hb[q], slin[q])
            pltpu.async_copy(g_hbm.at[pl.ds(base, CH)], gb[q], slin[q])
            pltpu.async_copy(e_hbm.at[pl.ds(base, CH)], eb[q], slin[q])

        def wait_lin(q):
            pltpu.make_async_copy(head_hbm.at[pl.ds(0, CH)], hb[q], slin[q]).wait()
            pltpu.make_async_copy(g_hbm.at[pl.ds(0, CH)], gb[q], slin[q]).wait()
            pltpu.make_async_copy(e_hbm.at[pl.ds(0, CH)], eb[q], slin[q]).wait()

        def issue_gather(q):
            @pl.when(c == 0)
            def _():
                pltpu.async_copy(lo_hbm.at[gb[q]], rows[q], srow[q])

            @pl.when(c == 1)
            def _():
                pltpu.async_copy(hi_hbm.at[gb[q]], rows[q], srow[q])

        def wait_gather(q):
            pltpu.make_async_copy(
                lo_hbm.at[pl.ds(0, CH)], rows[q], srow[q]).wait()

        def main(q, j):
            cid, real = cid_of(j)
            realf = real.astype(jnp.float32)
            for k in range(KV):
                sv = eb[q][pl.ds(k * L, L)] * realf
                for i in range(L):
                    row = k * L + i
                    sp = sv[i]
                    rows[q][row, 0:L] = rows[q][row, 0:L] * sp
                    rows[q][row, L:2 * L] = rows[q][row, L:2 * L] * sp
            pltpu.sync_copy(rows[q], agg.at[hb[q]], add=True)

        issue_lin(0, 0)
        issue_lin(1, 1)

        def pair(jj, _):
            j0 = 2 * jj
            wait_lin(0)
            issue_gather(0)
            wait_lin(1)
            issue_gather(1)
            wait_gather(0)
            main(0, j0)
            issue_lin(0, j0 + 2)
            wait_gather(1)
            main(1, j0 + 1)
            issue_lin(1, j0 + 3)
            return 0

        lax.fori_loop(0, np_, pair, 0)
        for q in (0, 1):
            wait_lin(q)
        plsc.subcore_barrier()

        def drain(q, _):
            pltpu.sync_copy(agg.at[pl.ds(off + q * ZR, ZR)], st_b)

            @pl.when(c == 0)
            def _():
                pltpu.sync_copy(st_b, out_lo.at[pl.ds(off + q * ZR, ZR)])

            @pl.when(c == 1)
            def _():
                pltpu.sync_copy(st_b, out_hi.at[pl.ds(off + q * ZR, ZR)])

            return 0

        lax.fori_loop(0, SLICE // ZR, drain, 0)

    cb_i = pltpu.VMEM((CH,), jnp.int32)
    cb_f = pltpu.VMEM((CH,), jnp.float32)
    rows_t = pltpu.VMEM((CH, 32), jnp.float32)
    return pl.kernel(
        body,
        out_type=(
            jax.ShapeDtypeStruct((NPAD, 32), jnp.float32),
            jax.ShapeDtypeStruct((NPAD, 32), jnp.float32),
        ),
        mesh=mesh,
        compiler_params=_SC_PARAMS,
        scratch_types=[
            cb_i, cb_i, cb_f, rows_t,
            cb_i, cb_i, cb_f, rows_t,
            pltpu.VMEM((ZR, 32), jnp.float32),
            pltpu.VMEM((ZR, 32), jnp.float32),
            pltpu.VMEM_SHARED((NPAD, 32), jnp.float32),
            pltpu.SemaphoreType.DMA, pltpu.SemaphoreType.DMA,
            pltpu.SemaphoreType.DMA, pltpu.SemaphoreType.DMA,
        ],
    )


def _make_final_body(n_ent, eb, grid):
    tail = n_ent - (grid - 1) * eb

    def body(im_ref, ent_ref, u_ref, lo_ref, hi_ref, d0_ref, d1_ref,
             eagg_ref, uagg_ref):
        i = pl.program_id(0)
        im = im_ref[...]
        ent = ent_ref[...]

        @pl.when(i == grid - 1)
        def _():
            cols = lax.broadcasted_iota(jnp.int32, im.shape, 1)
            rows = lax.broadcasted_iota(jnp.int32, ent.shape, 0)
            im_ref[...] = jnp.where(cols < tail, im, 0.0)
            ent_ref[...] = jnp.where(rows < tail, ent, 0.0)

        imz = im_ref[...]
        base = lax.dot_general(imz, u_ref[...], (((0,), (0,)), ((), ())),
                               preferred_element_type=jnp.float32)
        d = d0_ref[...] + d1_ref[...]
        dinv = 1.0 / jnp.where(d > 0.0, d, 1.0)
        eagg_ref[:, 0:32] = lo_ref[...] * dinv + base[:, 0:32]
        eagg_ref[:, 32:64] = hi_ref[...] * dinv + base[:, 32:64]

        @pl.when(i == 0)
        def _():
            uagg_ref[...] = jnp.zeros_like(uagg_ref)

        uagg_ref[...] += jnp.dot(imz, ent_ref[...],
                                 preferred_element_type=jnp.float32)

    return body


def _tc_final(interact_mat, entity_emb, user_emb, sc_lo, sc_hi, d0, d1):
    n_usr, n_ent = interact_mat.shape
    d = entity_emb.shape[1]
    eb = 2048
    grid = (n_ent + eb - 1) // eb
    return pl.pallas_call(
        _make_final_body(n_ent, eb, grid),
        grid=(grid,),
        in_specs=[
            pl.BlockSpec((n_usr, eb), lambda i: (0, i)),
            pl.BlockSpec((eb, d), lambda i: (i, 0)),
            pl.BlockSpec((n_usr, d), lambda i: (0, 0)),
            pl.BlockSpec((eb, 32), lambda i: (i, 0)),
            pl.BlockSpec((eb, 32), lambda i: (i, 0)),
            pl.BlockSpec((eb, 1), lambda i: (i, 0)),
            pl.BlockSpec((eb, 1), lambda i: (i, 0)),
        ],
        out_specs=[
            pl.BlockSpec((eb, d), lambda i: (i, 0)),
            pl.BlockSpec((n_usr, d), lambda i: (0, 0)),
        ],
        out_shape=[
            jax.ShapeDtypeStruct((n_ent, d), jnp.float32),
            jax.ShapeDtypeStruct((n_usr, d), jnp.float32),
        ],
        compiler_params=pltpu.CompilerParams(
            dimension_semantics=("arbitrary",)),
    )(interact_mat, entity_emb, user_emb, sc_lo, sc_hi, d0, d1)


def kernel(entity_emb, user_emb, edge_index, edge_type, interact_mat, weight):
    n_ent = entity_emb.shape[0]
    n_edge = edge_index.shape[1]
    head = edge_index[0]
    tail = edge_index[1]

    lo, hi, nt = _tc_prep(entity_emb, weight)
    a_e, g_idx, m_sc, _ = _make_sc1(n_ent, n_edge)(
        nt.reshape(-1), head, tail, edge_type)
    e_e, d_sc, _ = _make_sc2(n_ent, n_edge)(head, a_e, m_sc)
    sc_lo, sc_hi = _make_sc3(n_ent, n_edge)(lo, hi, head, g_idx, e_e)
    entity_agg, user_agg = _tc_final(
        interact_mat, entity_emb, user_emb,
        sc_lo[:n_ent], sc_hi[:n_ent],
        d_sc[:n_ent].reshape(n_ent, 1),
        d_sc[NPAD:NPAD + n_ent].reshape(n_ent, 1))
    return (entity_agg, user_agg)


# speedup vs baseline: 11.3470x; 1.3343x over previous
import jax
import jax.numpy as jnp
from jax import lax
from jax.experimental import pallas as pl
from jax.experimental.pallas import tpu as pltpu
from jax.experimental.pallas import tpu_sc as plsc

NC, NS, L = 2, 16, 16
NW = NC * NS
CH = 128
KV = CH // L
SLICE = 3136
NPAD = SLICE * NS
NVS = SLICE // L
MCH = 2000
ZR = 112

_SC_PARAMS = pltpu.CompilerParams(
    needs_layout_passes=False, use_tc_tiling_on_sc=False)

_GDN = lax.GatherDimensionNumbers(
    offset_dims=(), collapsed_slice_dims=(0,), start_index_map=(0,))


def _vgather(x, idx):
    return lax.gather(x, idx[:, None], _GDN, (1,),
                      mode=lax.GatherScatterMode.PROMISE_IN_BOUNDS)


def _seg_update(m_ref, lane_t, idx, val):
    iota = lax.iota(jnp.int32, L)
    fiota = iota.astype(jnp.float32)
    plsc.store_scatter(lane_t, [idx], fiota, mask=idx >= 0)
    got = plsc.load_gather(lane_t, [idx])
    cur = plsc.load_gather(m_ref, [idx])
    leader = got == fiota
    has_dup = jnp.any(jnp.logical_not(leader))

    @pl.when(jnp.logical_not(has_dup))
    def _():
        plsc.store_scatter(m_ref, [idx], val, mask=val > cur)

    @pl.when(has_dup)
    def _():
        vmax = val
        for d in range(1, L):
            src = (iota + d) & (L - 1)
            oi = _vgather(idx, src)
            ov = _vgather(val, src)
            vmax = jnp.where(oi == idx, jnp.maximum(vmax, ov), vmax)
        plsc.store_scatter(m_ref, [idx], vmax, mask=leader & (vmax > cur))


def _merge_slice(part_hbm, out_ref, core, sid, acc_b, stg_b, combine):
    off = sid * SLICE
    row0 = core * NS * NPAD
    pltpu.sync_copy(part_hbm.at[pl.ds(row0 + off, SLICE)], acc_b)

    def one_row(j, _):
        pltpu.sync_copy(part_hbm.at[pl.ds(row0 + j * NPAD + off, SLICE)], stg_b)

        def one_vreg(q, _):
            sl = pl.ds(q * L, L)
            acc_b[sl] = combine(acc_b[sl], stg_b[sl])
            return 0

        return lax.fori_loop(0, NVS, one_vreg, 0)

    lax.fori_loop(1, NS, one_row, 0)
    pltpu.sync_copy(acc_b, out_ref.at[pl.ds(core * NPAD + off, SLICE)])


def _load_merged(src_ref, dst_ref, stg_b, combine, n):
    pltpu.sync_copy(src_ref.at[pl.ds(0, n)], dst_ref.at[pl.ds(0, n)])

    def one_chunk(p, _):
        pltpu.sync_copy(src_ref.at[pl.ds(NPAD + p * MCH, MCH)], stg_b)

        def one_vreg(q, _):
            sl = pl.ds(p * MCH + q * L, L)
            dst_ref[sl] = combine(dst_ref[sl], stg_b[pl.ds(q * L, L)])
            return 0

        return lax.fori_loop(0, MCH // L, one_vreg, 0)

    lax.fori_loop(0, n // MCH, one_chunk, 0)


def _fill(ref, n, value):
    vec = jnp.full((L,), value, ref.dtype)

    def one(i, _):
        ref[pl.ds(i * L, L)] = vec
        return 0

    lax.fori_loop(0, n // L, one, 0)


def _prep_body(ent_ref, w_ref, nt_ref):
    ent = ent_ref[...]
    w = w_ref[...]
    nt_ref[...] = jnp.dot(
        ent * ent, (w * w).T,
        preferred_element_type=jnp.float32,
        precision=lax.Precision.HIGHEST)


def _tc_prep(entity_emb, weight):
    n_ent, d = entity_emb.shape
    n_rel = weight.shape[0]
    rb = 400
    grid = n_ent // rb
    return pl.pallas_call(
        _prep_body,
        grid=(grid,),
        in_specs=[
            pl.BlockSpec((rb, d), lambda i: (i, 0)),
            pl.BlockSpec((n_rel, d), lambda i: (0, 0)),
        ],
        out_specs=[
            pl.BlockSpec((rb, n_rel), lambda i: (i, 0)),
        ],
        out_shape=[
            jax.ShapeDtypeStruct((n_ent, n_rel), jnp.float32),
        ],
    )(entity_emb, weight)


def _make_sc0(n_ent):
    EC = 8
    ncht = n_ent // EC
    nch_u = -(-ncht // NW)
    nch_u += nch_u % 2
    np_ = nch_u // 2
    rows_c = EC * 16
    mesh = plsc.VectorSubcoreMesh(core_axis_name="c", subcore_axis_name="s")

    def body(ent_hbm, w_hbm, lo_hbm, hi_hbm,
             w_b, e0, lo0, hi0, e1, lo1, hi1,
             sl0, sl1, ss0, ss1):
        c = lax.axis_index("c")
        s = lax.axis_index("s")
        wid = c * NS + s
        pltpu.sync_copy(w_hbm, w_b)
        ebuf = (e0, e1)
        lob = (lo0, lo1)
        hib = (hi0, hi1)
        slin = (sl0, sl1)
        sst = (ss0, ss1)

        def cid_of(j):
            return jnp.minimum(wid + NW * j, ncht - 1)

        def issue_lin(q, j):
            base = cid_of(j) * (EC * 64)
            pltpu.async_copy(ent_hbm.at[pl.ds(base, EC * 64)], ebuf[q], slin[q])

        def wait_lin(q):
            pltpu.make_async_copy(
                ent_hbm.at[pl.ds(0, EC * 64)], ebuf[q], slin[q]).wait()

        def drain_store(q):
            pltpu.make_async_copy(lob[q], lo_hbm.at[pl.ds(0, rows_c)], sst[q]).wait()
            pltpu.make_async_copy(hib[q], hi_hbm.at[pl.ds(0, rows_c)], sst[q]).wait()

        def step(q, j, jj):
            wait_lin(q)

            @pl.when(jj > 0)
            def _():
                drain_store(q)

            for v in range(EC):
                ev = [ebuf[q][pl.ds(v * 64 + dd * L, L)] for dd in range(4)]
                for r in range(16):
                    row = v * 16 + r
                    for dd in range(4):
                        wv = w_b[pl.ds(r * 64 + dd * L, L)]
                        dst = lob[q] if dd < 2 else hib[q]
                        dst[row, pl.ds((dd % 2) * L, L)] = ev[dd] * wv
            base = cid_of(j) * rows_c
            pltpu.async_copy(lob[q], lo_hbm.at[pl.ds(base, rows_c)], sst[q])
            pltpu.async_copy(hib[q], hi_hbm.at[pl.ds(base, rows_c)], sst[q])
            issue_lin(q, j + 2)

        issue_lin(0, 0)
        issue_lin(1, 1)

        def pair(jj, _):
            step(0, 2 * jj, jj)
            step(1, 2 * jj + 1, jj)
            return 0

        lax.fori_loop(0, np_, pair, 0)
        for q in (0, 1):
            drain_store(q)
            wait_lin(q)

    rows_t = pltpu.VMEM((rows_c, 32), jnp.float32)
    return pl.kernel(
        body,
        out_type=(
            jax.ShapeDtypeStruct((n_ent * 16, 32), jnp.float32),
            jax.ShapeDtypeStruct((n_ent * 16, 32), jnp.float32),
        ),
        mesh=mesh,
        compiler_params=_SC_PARAMS,
        scratch_types=[
            pltpu.VMEM((1024,), jnp.float32),
            pltpu.VMEM((EC * 64,), jnp.float32), rows_t, rows_t,
            pltpu.VMEM((EC * 64,), jnp.float32), rows_t, rows_t,
            pltpu.SemaphoreType.DMA, pltpu.SemaphoreType.DMA,
            pltpu.SemaphoreType.DMA, pltpu.SemaphoreType.DMA,
        ],
    )


def _make_sc1(n_ent, n_edge):
    ncht = n_edge // CH
    nch_u = -(-ncht // NW)
    nch_u += nch_u % 2
    np_ = nch_u // 2
    mesh = plsc.VectorSubcoreMesh(core_axis_name="c", subcore_axis_name="s")

    def body(nt_hbm, head_hbm, tail_hbm, et_hbm,
             a_hbm, g_hbm, msc_hbm, mpart_hbm,
             m_t, lane_t,
             h0, t0, r0, gh0, gt0, nh0, ntl0, a0,
             h1, t1, r1, gh1, gt1, nh1, ntl1, a1,
             acc_b, stg_b,
             sl0, sl1, sg0, sg1, ss0, ss1):
        c = lax.axis_index("c")
        s = lax.axis_index("s")
        wid = c * NS + s
        _fill(m_t, n_ent, -1.0)
        hb = (h0, h1)
        tb = (t0, t1)
        rb = (r0, r1)
        ghb = (gh0, gh1)
        gtb = (gt0, gt1)
        nhb = (nh0, nh1)
        ntlb = (ntl0, ntl1)
        ab = (a0, a1)
        slin = (sl0, sl1)
        sgat = (sg0, sg1)
        sst = (ss0, ss1)

        def cid_of(j):
            raw = wid + NW * j
            real = raw < ncht
            return jnp.minimum(raw, ncht - 1), real

        def issue_lin(q, j):
            cid, _ = cid_of(j)
            base = cid * CH
            pltpu.async_copy(head_hbm.at[pl.ds(base, CH)], hb[q], slin[q])
            pltpu.async_copy(tail_hbm.at[pl.ds(base, CH)], tb[q], slin[q])
            pltpu.async_copy(et_hbm.at[pl.ds(base, CH)], rb[q], slin[q])

        def wait_lin(q):
            pltpu.make_async_copy(head_hbm.at[pl.ds(0, CH)], hb[q], slin[q]).wait()
            pltpu.make_async_copy(tail_hbm.at[pl.ds(0, CH)], tb[q], slin[q]).wait()
            pltpu.make_async_copy(et_hbm.at[pl.ds(0, CH)], rb[q], slin[q]).wait()

        def drain_store(q):
            pltpu.make_async_copy(ab[q], a_hbm.at[pl.ds(0, CH)], sst[q]).wait()
            pltpu.make_async_copy(gtb[q], g_hbm.at[pl.ds(0, CH)], sst[q]).wait()

        def idx_and_gather(q):
            for k in range(KV):
                sl = pl.ds(k * L, L)
                ridx = (rb[q][sl] - 1) & 15
                ghb[q][sl] = hb[q][sl] * 16 + ridx
                gtb[q][sl] = tb[q][sl] * 16 + ridx
            pltpu.async_copy(nt_hbm.at[ghb[q]], nhb[q], sgat[q])
            pltpu.async_copy(nt_hbm.at[gtb[q]], ntlb[q], sgat[q])

        def wait_gather(q):
            pltpu.make_async_copy(nt_hbm.at[pl.ds(0, CH)], nhb[q], sgat[q]).wait()
            pltpu.make_async_copy(nt_hbm.at[pl.ds(0, CH)], ntlb[q], sgat[q]).wait()

        def main(q, j):
            cid, real = cid_of(j)
            realf = real.astype(jnp.float32)
            for k in range(KV):
                sl = pl.ds(k * L, L)
                a = nhb[q][sl] * ntlb[q][sl] * realf - (1.0 - realf)
                ab[q][sl] = a
                _seg_update(m_t, lane_t, hb[q][sl], a)
            base = jnp.where(real, cid * CH, n_edge)
            pltpu.async_copy(ab[q], a_hbm.at[pl.ds(base, CH)], sst[q])
            pltpu.async_copy(gtb[q], g_hbm.at[pl.ds(base, CH)], sst[q])

        for q in (0, 1):
            pltpu.async_copy(ab[q], a_hbm.at[pl.ds(n_edge, CH)], sst[q])
            pltpu.async_copy(gtb[q], g_hbm.at[pl.ds(n_edge, CH)], sst[q])
        issue_lin(0, 0)
        issue_lin(1, 1)

        def pair(jj, _):
            j0 = 2 * jj
            drain_store(0)
            wait_lin(0)
            idx_and_gather(0)
            drain_store(1)
            wait_lin(1)
            idx_and_gather(1)
            wait_gather(0)
            main(0, j0)
            issue_lin(0, j0 + 2)
            wait_gather(1)
            main(1, j0 + 3 - 2)
            issue_lin(1, j0 + 3)
            return 0

        lax.fori_loop(0, np_, pair, 0)
        for q in (0, 1):
            drain_store(q)
            wait_lin(q)

        pltpu.sync_copy(m_t, mpart_hbm.at[pl.ds(wid * NPAD, n_ent)])
        plsc.subcore_barrier()
        _merge_slice(mpart_hbm, msc_hbm, c, s, acc_b, stg_b, jnp.maximum)

    cb_i = pltpu.VMEM((CH,), jnp.int32)
    cb_f = pltpu.VMEM((CH,), jnp.float32)
    return pl.kernel(
        body,
        out_type=(
            jax.ShapeDtypeStruct((n_edge + CH,), jnp.float32),
            jax.ShapeDtypeStruct((n_edge + CH,), jnp.int32),
            jax.ShapeDtypeStruct((NC * NPAD,), jnp.float32),
            jax.ShapeDtypeStruct((NW * NPAD,), jnp.float32),
        ),
        mesh=mesh,
        compiler_params=_SC_PARAMS,
        scratch_types=[
            pltpu.VMEM((n_ent,), jnp.float32),
            pltpu.VMEM((n_ent,), jnp.float32),
            cb_i, cb_i, cb_i, cb_i, cb_i, cb_f, cb_f, cb_f,
            cb_i, cb_i, cb_i, cb_i, cb_i, cb_f, cb_f, cb_f,
            pltpu.VMEM((SLICE,), jnp.float32),
            pltpu.VMEM((SLICE,), jnp.float32),
            pltpu.SemaphoreType.DMA, pltpu.SemaphoreType.DMA,
            pltpu.SemaphoreType.DMA, pltpu.SemaphoreType.DMA,
            pltpu.SemaphoreType.DMA, pltpu.SemaphoreType.DMA,
        ],
    )


def _make_sc2(n_ent, n_edge):
    ncht = n_edge // CH
    nch_u = -(-ncht // NW)
    nch_u += nch_u % 2
    np_ = nch_u // 2
    mesh = plsc.VectorSubcoreMesh(core_axis_name="c", subcore_axis_name="s")

    def body(head_hbm, a_hbm, msc_hbm,
             e_hbm, dsc_hbm, dpart_hbm,
             m_t, d_t,
             h0, a0, e0, h1, a1, e1,
             mstg_b, acc_b, stg_b,
             sl0, sl1, ss0, ss1):
        c = lax.axis_index("c")
        s = lax.axis_index("s")
        wid = c * NS + s
        _load_merged(msc_hbm, m_t, mstg_b, jnp.maximum, n_ent)
        _fill(d_t, n_ent, 0.0)
        hb = (h0, h1)
        ab = (a0, a1)
        eb = (e0, e1)
        slin = (sl0, sl1)
        sst = (ss0, ss1)

        def cid_of(j):
            raw = wid + NW * j
            real = raw < ncht
            return jnp.minimum(raw, ncht - 1), real

        def issue_lin(q, j):
            cid, _ = cid_of(j)
            base = cid * CH
            pltpu.async_copy(head_hbm.at[pl.ds(base, CH)], hb[q], slin[q])
            pltpu.async_copy(a_hbm.at[pl.ds(base, CH)], ab[q], slin[q])

        def step(q, j):
            pltpu.make_async_copy(eb[q], e_hbm.at[pl.ds(0, CH)], sst[q]).wait()
            pltpu.make_async_copy(head_hbm.at[pl.ds(0, CH)], hb[q], slin[q]).wait()
            pltpu.make_async_copy(a_hbm.at[pl.ds(0, CH)], ab[q], slin[q]).wait()
            cid, real = cid_of(j)
            realf = real.astype(jnp.float32)
            for k in range(KV):
                sl = pl.ds(k * L, L)
                h = hb[q][sl]
                mv = plsc.load_gather(m_t, [h])
                e = jnp.exp(ab[q][sl] - mv) * realf
                eb[q][sl] = e
                plsc.addupdate_scatter(d_t, [h], e)
            base = jnp.where(real, cid * CH, n_edge)
            pltpu.async_copy(eb[q], e_hbm.at[pl.ds(base, CH)], sst[q])
            issue_lin(q, j + 2)

        for q in (0, 1):
            pltpu.async_copy(eb[q], e_hbm.at[pl.ds(n_edge, CH)], sst[q])
        issue_lin(0, 0)
        issue_lin(1, 1)

        def pair(jj, _):
            step(0, 2 * jj)
            step(1, 2 * jj + 1)
            return 0

        lax.fori_loop(0, np_, pair, 0)
        for q in (0, 1):
            pltpu.make_async_copy(eb[q], e_hbm.at[pl.ds(0, CH)], sst[q]).wait()
            pltpu.make_async_copy(head_hbm.at[pl.ds(0, CH)], hb[q], slin[q]).wait()
            pltpu.make_async_copy(a_hbm.at[pl.ds(0, CH)], ab[q], slin[q]).wait()

        pltpu.sync_copy(d_t, dpart_hbm.at[pl.ds(wid * NPAD, n_ent)])
        plsc.subcore_barrier()
        _merge_slice(dpart_hbm, dsc_hbm, c, s, acc_b, stg_b, jnp.add)

    cb_i = pltpu.VMEM((CH,), jnp.int32)
    cb_f = pltpu.VMEM((CH,), jnp.float32)
    return pl.kernel(
        body,
        out_type=(
            jax.ShapeDtypeStruct((n_edge + CH,), jnp.float32),
            jax.ShapeDtypeStruct((NC * NPAD,), jnp.float32),
            jax.ShapeDtypeStruct((NW * NPAD,), jnp.float32),
        ),
        mesh=mesh,
        compiler_params=_SC_PARAMS,
        scratch_types=[
            pltpu.VMEM((n_ent,), jnp.float32),
            pltpu.VMEM((n_ent,), jnp.float32),
            cb_i, cb_f, cb_f, cb_i, cb_f, cb_f,
            pltpu.VMEM((MCH,), jnp.float32),
            pltpu.VMEM((SLICE,), jnp.float32),
            pltpu.VMEM((SLICE,), jnp.float32),
            pltpu.SemaphoreType.DMA, pltpu.SemaphoreType.DMA,
            pltpu.SemaphoreType.DMA, pltpu.SemaphoreType.DMA,
        ],
    )


def _make_sc3(n_ent, n_edge):
    ncht = n_edge // CH
    nch_u = -(-ncht // NS)
    nch_u += nch_u % 2
    np_ = nch_u // 2
    mesh = plsc.VectorSubcoreMesh(core_axis_name="c", subcore_axis_name="s")

    def body(lo_hbm, hi_hbm, head_hbm, g_hbm, e_hbm,
             out_lo, out_hi,
             h0, g0, e0, rows0, h1, g1, e1, rows1,
             z_b, st_b, agg,
             sl0, sl1, sr0, sr1):
        c = lax.axis_index("c")
        s = lax.axis_index("s")
        hb = (h0, h1)
        gb = (g0, g1)
        eb = (e0, e1)
        rows = (rows0, rows1)
        slin = (sl0, sl1)
        srow = (sr0, sr1)

        for i in range(ZR):
            for j in range(2):
                z_b[i, pl.ds(j * L, L)] = jnp.zeros((L,), jnp.float32)
        off = s * SLICE

        def zloop(q, _):
            pltpu.sync_copy(z_b, agg.at[pl.ds(off + q * ZR, ZR)])
            return 0

        lax.fori_loop(0, SLICE // ZR, zloop, 0)
        plsc.subcore_barrier()

        def cid_of(j):
            raw = s + NS * j
            real = raw < ncht
            return jnp.minimum(raw, ncht - 1), real

        def issue_lin(q, j):
            cid, _ = cid_of(j)
            base = cid * CH
            pltpu.async_copy(head_hbm.at[pl.ds(base, CH)], hb[q], slin[q])
            pltpu.async_copy(g_hbm.at[pl.ds(base, CH)], gb[q], slin[q])
            pltpu.async_copy(e_hbm.at[pl.ds(base, CH)], eb[q], slin[q])

        def wait_lin(q):
            pltpu.make_async_copy(head_hbm.at[pl.ds(0, CH)], hb[q], slin[q]).wait()
            pltpu.make_async_copy(g_hbm.at[pl.ds(0, CH)], gb[q], slin[q]).wait()
            pltpu.make_async_copy(e_hbm.at[pl.ds(0, CH)], eb[q], slin[q]).wait()

        def issue_gather(q):
            @pl.when(c == 0)
            def _():
                pltpu.async_copy(lo_hbm.at[gb[q]], rows[q], srow[q])

            @pl.when(c == 1)
            def _():
                pltpu.async_copy(hi_hbm.at[gb[q]], rows[q], srow[q])

        def wait_gather(q):
            pltpu.make_async_copy(
                lo_hbm.at[pl.ds(0, CH)], rows[q], srow[q]).wait()

        def main(q, j):
            cid, real = cid_of(j)
            realf = real.astype(jnp.float32)
            for k in range(KV):
                sv = eb[q][pl.ds(k * L, L)] * realf
                for i in range(L):
                    row = k * L + i
                    sp = sv[i]
                    rows[q][row, 0:L] = rows[q][row, 0:L] * sp
                    rows[q][row, L:2 * L] = rows[q][row, L:2 * L] * sp
            pltpu.sync_copy(rows[q], agg.at[hb[q]], add=True)

        issue_lin(0, 0)
        issue_lin(1, 1)

        def pair(jj, _):
            j0 = 2 * jj
            wait_lin(0)
            issue_gather(0)
            wait_lin(1)
            issue_gather(1)
            wait_gather(0)
            main(0, j0)
            issue_lin(0, j0 + 2)
            wait_gather(1)
            main(1, j0 + 1)
            issue_lin(1, j0 + 3)
            return 0

        lax.fori_loop(0, np_, pair, 0)
        for q in (0, 1):
            wait_lin(q)
        plsc.subcore_barrier()

        def drain(q, _):
            pltpu.sync_copy(agg.at[pl.ds(off + q * ZR, ZR)], st_b)

            @pl.when(c == 0)
            def _():
                pltpu.sync_copy(st_b, out_lo.at[pl.ds(off + q * ZR, ZR)])

            @pl.when(c == 1)
            def _():
                pltpu.sync_copy(st_b, out_hi.at[pl.ds(off + q * ZR, ZR)])

            return 0

        lax.fori_loop(0, SLICE // ZR, drain, 0)

    cb_i = pltpu.VMEM((CH,), jnp.int32)
    cb_f = pltpu.VMEM((CH,), jnp.float32)
    rows_t = pltpu.VMEM((CH, 32), jnp.float32)
    return pl.kernel(
        body,
        out_type=(
            jax.ShapeDtypeStruct((NPAD, 32), jnp.float32),
            jax.ShapeDtypeStruct((NPAD, 32), jnp.float32),
        ),
        mesh=mesh,
        compiler_params=_SC_PARAMS,
        scratch_types=[
            cb_i, cb_i, cb_f, rows_t,
            cb_i, cb_i, cb_f, rows_t,
            pltpu.VMEM((ZR, 32), jnp.float32),
            pltpu.VMEM((ZR, 32), jnp.float32),
            pltpu.VMEM_SHARED((NPAD, 32), jnp.float32),
            pltpu.SemaphoreType.DMA, pltpu.SemaphoreType.DMA,
            pltpu.SemaphoreType.DMA, pltpu.SemaphoreType.DMA,
        ],
    )


def _make_final_body(n_ent, eb, grid):
    tail = n_ent - (grid - 1) * eb

    def body(im_ref, ent_ref, u_ref, lo_ref, hi_ref, d0_ref, d1_ref,
             eagg_ref, uagg_ref):
        i = pl.program_id(0)
        im = im_ref[...]
        ent = ent_ref[...]

        @pl.when(i == grid - 1)
        def _():
            cols = lax.broadcasted_iota(jnp.int32, im.shape, 1)
            rows = lax.broadcasted_iota(jnp.int32, ent.shape, 0)
            im_ref[...] = jnp.where(cols < tail, im, 0.0)
            ent_ref[...] = jnp.where(rows < tail, ent, 0.0)

        imz = im_ref[...]
        base = lax.dot_general(imz, u_ref[...], (((0,), (0,)), ((), ())),
                               preferred_element_type=jnp.float32)
        d = d0_ref[...] + d1_ref[...]
        dinv = 1.0 / jnp.where(d > 0.0, d, 1.0)
        eagg_ref[:, 0:32] = lo_ref[...] * dinv + base[:, 0:32]
        eagg_ref[:, 32:64] = hi_ref[...] * dinv + base[:, 32:64]

        @pl.when(i == 0)
        def _():
            uagg_ref[...] = jnp.zeros_like(uagg_ref)

        uagg_ref[...] += jnp.dot(imz, ent_ref[...],
                                 preferred_element_type=jnp.float32)

    return body


def _tc_final(interact_mat, entity_emb, user_emb, sc_lo, sc_hi, d0, d1):
    n_usr, n_ent = interact_mat.shape
    d = entity_emb.shape[1]
    eb = 2048
    grid = (n_ent + eb - 1) // eb
    return pl.pallas_call(
        _make_final_body(n_ent, eb, grid),
        grid=(grid,),
        in_specs=[
            pl.BlockSpec((n_usr, eb), lambda i: (0, i)),
            pl.BlockSpec((eb, d), lambda i: (i, 0)),
            pl.BlockSpec((n_usr, d), lambda i: (0, 0)),
            pl.BlockSpec((eb, 32), lambda i: (i, 0)),
            pl.BlockSpec((eb, 32), lambda i: (i, 0)),
            pl.BlockSpec((eb, 1), lambda i: (i, 0)),
            pl.BlockSpec((eb, 1), lambda i: (i, 0)),
        ],
        out_specs=[
            pl.BlockSpec((eb, d), lambda i: (i, 0)),
            pl.BlockSpec((n_usr, d), lambda i: (0, 0)),
        ],
        out_shape=[
            jax.ShapeDtypeStruct((n_ent, d), jnp.float32),
            jax.ShapeDtypeStruct((n_usr, d), jnp.float32),
        ],
        compiler_params=pltpu.CompilerParams(
            dimension_semantics=("arbitrary",)),
    )(interact_mat, entity_emb, user_emb, sc_lo, sc_hi, d0, d1)


def kernel(entity_emb, user_emb, edge_index, edge_type, interact_mat, weight):
    n_ent = entity_emb.shape[0]
    n_edge = edge_index.shape[1]
    head = edge_index[0]
    tail = edge_index[1]

    lo, hi = _make_sc0(n_ent)(entity_emb.reshape(-1), weight.reshape(-1))
    (nt2,) = _tc_prep(entity_emb, weight)
    a_e, g_idx, m_sc, _ = _make_sc1(n_ent, n_edge)(
        nt2.reshape(-1), head, tail, edge_type)
    e_e, d_sc, _ = _make_sc2(n_ent, n_edge)(head, a_e, m_sc)
    sc_lo, sc_hi = _make_sc3(n_ent, n_edge)(lo, hi, head, g_idx, e_e)
    entity_agg, user_agg = _tc_final(
        interact_mat, entity_emb, user_emb,
        sc_lo[:n_ent], sc_hi[:n_ent],
        d_sc[:n_ent].reshape(n_ent, 1),
        d_sc[NPAD:NPAD + n_ent].reshape(n_ent, 1))
    return (entity_agg, user_agg)


# speedup vs baseline: 12.4658x; 1.0986x over previous
import jax
import jax.numpy as jnp
from jax import lax
from jax.experimental import pallas as pl
from jax.experimental.pallas import tpu as pltpu
from jax.experimental.pallas import tpu_sc as plsc

NC, NS, L = 2, 16, 16
NW = NC * NS
CH = 128
KV = CH // L
SLICE = 3200
NPAD = SLICE * NS
NVS = SLICE // L
MCH = 2000
ZR = 128

_SC_PARAMS = pltpu.CompilerParams(
    needs_layout_passes=False, use_tc_tiling_on_sc=False)

_GDN = lax.GatherDimensionNumbers(
    offset_dims=(), collapsed_slice_dims=(0,), start_index_map=(0,))


def _vgather(x, idx):
    return lax.gather(x, idx[:, None], _GDN, (1,),
                      mode=lax.GatherScatterMode.PROMISE_IN_BOUNDS)


def _seg_update(m_ref, lane_t, idx, val):
    iota = lax.iota(jnp.int32, L)
    fiota = iota.astype(jnp.float32)
    plsc.store_scatter(lane_t, [idx], fiota, mask=idx >= 0)
    got = plsc.load_gather(lane_t, [idx])
    cur = plsc.load_gather(m_ref, [idx])
    leader = got == fiota
    has_dup = jnp.any(jnp.logical_not(leader))

    @pl.when(jnp.logical_not(has_dup))
    def _():
        plsc.store_scatter(m_ref, [idx], val, mask=val > cur)

    @pl.when(has_dup)
    def _():
        vmax = val
        for d in range(1, L):
            src = (iota + d) & (L - 1)
            oi = _vgather(idx, src)
            ov = _vgather(val, src)
            vmax = jnp.where(oi == idx, jnp.maximum(vmax, ov), vmax)
        plsc.store_scatter(m_ref, [idx], vmax, mask=leader & (vmax > cur))


def _merge_slice(part_hbm, out_ref, core, sid, acc_b, stg_b, combine):
    off = sid * SLICE
    row0 = core * NS * NPAD
    pltpu.sync_copy(part_hbm.at[pl.ds(row0 + off, SLICE)], acc_b)

    def one_row(j, _):
        pltpu.sync_copy(part_hbm.at[pl.ds(row0 + j * NPAD + off, SLICE)], stg_b)

        def one_vreg(q, _):
            sl = pl.ds(q * L, L)
            acc_b[sl] = combine(acc_b[sl], stg_b[sl])
            return 0

        return lax.fori_loop(0, NVS, one_vreg, 0)

    lax.fori_loop(1, NS, one_row, 0)
    pltpu.sync_copy(acc_b, out_ref.at[pl.ds(core * NPAD + off, SLICE)])


def _load_merged(src_ref, dst_ref, stg_b, combine, n):
    pltpu.sync_copy(src_ref.at[pl.ds(0, n)], dst_ref.at[pl.ds(0, n)])

    def one_chunk(p, _):
        pltpu.sync_copy(src_ref.at[pl.ds(NPAD + p * MCH, MCH)], stg_b)

        def one_vreg(q, _):
            sl = pl.ds(p * MCH + q * L, L)
            dst_ref[sl] = combine(dst_ref[sl], stg_b[pl.ds(q * L, L)])
            return 0

        return lax.fori_loop(0, MCH // L, one_vreg, 0)

    lax.fori_loop(0, n // MCH, one_chunk, 0)


def _fill(ref, n, value):
    vec = jnp.full((L,), value, ref.dtype)

    def one(i, _):
        ref[pl.ds(i * L, L)] = vec
        return 0

    lax.fori_loop(0, n // L, one, 0)


def _prep_body(ent_ref, w_ref, nt_ref):
    ent = ent_ref[...]
    w = w_ref[...]
    nt_ref[...] = jnp.dot(
        ent * ent, (w * w).T,
        preferred_element_type=jnp.float32,
        precision=lax.Precision.HIGHEST)


def _tc_prep(entity_emb, weight):
    n_ent, d = entity_emb.shape
    n_rel = weight.shape[0]
    rb = 2000
    grid = n_ent // rb
    return pl.pallas_call(
        _prep_body,
        grid=(grid,),
        in_specs=[
            pl.BlockSpec((rb, d), lambda i: (i, 0)),
            pl.BlockSpec((n_rel, d), lambda i: (0, 0)),
        ],
        out_specs=[
            pl.BlockSpec((rb, n_rel), lambda i: (i, 0)),
        ],
        out_shape=[
            jax.ShapeDtypeStruct((n_ent, n_rel), jnp.float32),
        ],
    )(entity_emb, weight)


def _make_sc0(n_ent):
    EC = 8
    ncht = n_ent // EC
    nch_u = -(-ncht // NW)
    nch_u += nch_u % 2
    np_ = nch_u // 2
    rows_c = EC * 16
    mesh = plsc.VectorSubcoreMesh(core_axis_name="c", subcore_axis_name="s")

    def body(ent_hbm, w_hbm, lo_hbm, hi_hbm,
             w_b, e0, lo0, hi0, e1, lo1, hi1,
             sl0, sl1, ss0, ss1):
        c = lax.axis_index("c")
        s = lax.axis_index("s")
        wid = c * NS + s
        pltpu.sync_copy(w_hbm, w_b)
        ebuf = (e0, e1)
        lob = (lo0, lo1)
        hib = (hi0, hi1)
        slin = (sl0, sl1)
        sst = (ss0, ss1)

        def cid_of(j):
            return jnp.minimum(wid + NW * j, ncht - 1)

        def issue_lin(q, j):
            base = cid_of(j) * (EC * 64)
            pltpu.async_copy(ent_hbm.at[pl.ds(base, EC * 64)], ebuf[q], slin[q])

        def wait_lin(q):
            pltpu.make_async_copy(
                ent_hbm.at[pl.ds(0, EC * 64)], ebuf[q], slin[q]).wait()

        def drain_store(q):
            pltpu.make_async_copy(lob[q], lo_hbm.at[pl.ds(0, rows_c)], sst[q]).wait()
            pltpu.make_async_copy(hib[q], hi_hbm.at[pl.ds(0, rows_c)], sst[q]).wait()

        def step(q, j, jj):
            wait_lin(q)

            @pl.when(jj > 0)
            def _():
                drain_store(q)

            for v in range(EC):
                ev = [ebuf[q][pl.ds(v * 64 + dd * L, L)] for dd in range(4)]
                for r in range(16):
                    row = v * 16 + r
                    for dd in range(4):
                        wv = w_b[pl.ds(r * 64 + dd * L, L)]
                        dst = lob[q] if dd < 2 else hib[q]
                        dst[row, pl.ds((dd % 2) * L, L)] = ev[dd] * wv
            base = cid_of(j) * rows_c
            pltpu.async_copy(lob[q], lo_hbm.at[pl.ds(base, rows_c)], sst[q])
            pltpu.async_copy(hib[q], hi_hbm.at[pl.ds(base, rows_c)], sst[q])
            issue_lin(q, j + 2)

        issue_lin(0, 0)
        issue_lin(1, 1)

        def pair(jj, _):
            step(0, 2 * jj, jj)
            step(1, 2 * jj + 1, jj)
            return 0

        lax.fori_loop(0, np_, pair, 0)
        for q in (0, 1):
            drain_store(q)
            wait_lin(q)

    rows_t = pltpu.VMEM((rows_c, 32), jnp.float32)
    return pl.kernel(
        body,
        out_type=(
            jax.ShapeDtypeStruct((n_ent * 16, 32), jnp.float32),
            jax.ShapeDtypeStruct((n_ent * 16, 32), jnp.float32),
        ),
        mesh=mesh,
        compiler_params=_SC_PARAMS,
        scratch_types=[
            pltpu.VMEM((1024,), jnp.float32),
            pltpu.VMEM((EC * 64,), jnp.float32), rows_t, rows_t,
            pltpu.VMEM((EC * 64,), jnp.float32), rows_t, rows_t,
            pltpu.SemaphoreType.DMA, pltpu.SemaphoreType.DMA,
            pltpu.SemaphoreType.DMA, pltpu.SemaphoreType.DMA,
        ],
    )


def _make_sc1(n_ent, n_edge):
    ncht = n_edge // CH
    nch_u = -(-ncht // NW)
    nch_u += nch_u % 2
    np_ = nch_u // 2
    mesh = plsc.VectorSubcoreMesh(core_axis_name="c", subcore_axis_name="s")

    def body(nt_hbm, head_hbm, tail_hbm, et_hbm,
             a_hbm, g_hbm, msc_hbm, mpart_hbm,
             m_t, lane_t,
             h0, t0, r0, gh0, gt0, nh0, ntl0, a0,
             h1, t1, r1, gh1, gt1, nh1, ntl1, a1,
             acc_b, stg_b,
             sl0, sl1, sg0, sg1, ss0, ss1):
        c = lax.axis_index("c")
        s = lax.axis_index("s")
        wid = c * NS + s
        _fill(m_t, n_ent, -1.0)
        hb = (h0, h1)
        tb = (t0, t1)
        rb = (r0, r1)
        ghb = (gh0, gh1)
        gtb = (gt0, gt1)
        nhb = (nh0, nh1)
        ntlb = (ntl0, ntl1)
        ab = (a0, a1)
        slin = (sl0, sl1)
        sgat = (sg0, sg1)
        sst = (ss0, ss1)

        def cid_of(j):
            raw = wid + NW * j
            real = raw < ncht
            return jnp.minimum(raw, ncht - 1), real

        def issue_lin(q, j):
            cid, _ = cid_of(j)
            base = cid * CH
            pltpu.async_copy(head_hbm.at[pl.ds(base, CH)], hb[q], slin[q])
            pltpu.async_copy(tail_hbm.at[pl.ds(base, CH)], tb[q], slin[q])
            pltpu.async_copy(et_hbm.at[pl.ds(base, CH)], rb[q], slin[q])

        def wait_lin(q):
            pltpu.make_async_copy(head_hbm.at[pl.ds(0, CH)], hb[q], slin[q]).wait()
            pltpu.make_async_copy(tail_hbm.at[pl.ds(0, CH)], tb[q], slin[q]).wait()
            pltpu.make_async_copy(et_hbm.at[pl.ds(0, CH)], rb[q], slin[q]).wait()

        def drain_store(q):
            pltpu.make_async_copy(ab[q], a_hbm.at[pl.ds(0, CH)], sst[q]).wait()
            pltpu.make_async_copy(gtb[q], g_hbm.at[pl.ds(0, CH)], sst[q]).wait()

        def idx_and_gather(q):
            for k in range(KV):
                sl = pl.ds(k * L, L)
                ridx = (rb[q][sl] - 1) & 15
                ghb[q][sl] = hb[q][sl] * 16 + ridx
                gtb[q][sl] = tb[q][sl] * 16 + ridx
            pltpu.async_copy(nt_hbm.at[ghb[q]], nhb[q], sgat[q])
            pltpu.async_copy(nt_hbm.at[gtb[q]], ntlb[q], sgat[q])

        def wait_gather(q):
            pltpu.make_async_copy(nt_hbm.at[pl.ds(0, CH)], nhb[q], sgat[q]).wait()
            pltpu.make_async_copy(nt_hbm.at[pl.ds(0, CH)], ntlb[q], sgat[q]).wait()

        def main(q, j):
            cid, real = cid_of(j)
            realf = real.astype(jnp.float32)
            for k in range(KV):
                sl = pl.ds(k * L, L)
                a = nhb[q][sl] * ntlb[q][sl] * realf - (1.0 - realf)
                ab[q][sl] = a
                _seg_update(m_t, lane_t, hb[q][sl], a)
            base = jnp.where(real, cid * CH, n_edge)
            pltpu.async_copy(ab[q], a_hbm.at[pl.ds(base, CH)], sst[q])
            pltpu.async_copy(gtb[q], g_hbm.at[pl.ds(base, CH)], sst[q])

        for q in (0, 1):
            pltpu.async_copy(ab[q], a_hbm.at[pl.ds(n_edge, CH)], sst[q])
            pltpu.async_copy(gtb[q], g_hbm.at[pl.ds(n_edge, CH)], sst[q])
        issue_lin(0, 0)
        issue_lin(1, 1)

        def pair(jj, _):
            j0 = 2 * jj
            drain_store(0)
            wait_lin(0)
            idx_and_gather(0)
            drain_store(1)
            wait_lin(1)
            idx_and_gather(1)
            wait_gather(0)
            main(0, j0)
            issue_lin(0, j0 + 2)
            wait_gather(1)
            main(1, j0 + 3 - 2)
            issue_lin(1, j0 + 3)
            return 0

        lax.fori_loop(0, np_, pair, 0)
        for q in (0, 1):
            drain_store(q)
            wait_lin(q)

        pltpu.sync_copy(m_t, mpart_hbm.at[pl.ds(wid * NPAD, n_ent)])
        plsc.subcore_barrier()
        _merge_slice(mpart_hbm, msc_hbm, c, s, acc_b, stg_b, jnp.maximum)

    cb_i = pltpu.VMEM((CH,), jnp.int32)
    cb_f = pltpu.VMEM((CH,), jnp.float32)
    return pl.kernel(
        body,
        out_type=(
            jax.ShapeDtypeStruct((n_edge + CH,), jnp.float32),
            jax.ShapeDtypeStruct((n_edge + CH,), jnp.int32),
            jax.ShapeDtypeStruct((NC * NPAD,), jnp.float32),
            jax.ShapeDtypeStruct((NW * NPAD,), jnp.float32),
        ),
        mesh=mesh,
        compiler_params=_SC_PARAMS,
        scratch_types=[
            pltpu.VMEM((n_ent,), jnp.float32),
            pltpu.VMEM((n_ent,), jnp.float32),
            cb_i, cb_i, cb_i, cb_i, cb_i, cb_f, cb_f, cb_f,
            cb_i, cb_i, cb_i, cb_i, cb_i, cb_f, cb_f, cb_f,
            pltpu.VMEM((SLICE,), jnp.float32),
            pltpu.VMEM((SLICE,), jnp.float32),
            pltpu.SemaphoreType.DMA, pltpu.SemaphoreType.DMA,
            pltpu.SemaphoreType.DMA, pltpu.SemaphoreType.DMA,
            pltpu.SemaphoreType.DMA, pltpu.SemaphoreType.DMA,
        ],
    )


def _make_sc2(n_ent, n_edge):
    ncht = n_edge // CH
    nch_u = -(-ncht // NW)
    nch_u += nch_u % 2
    np_ = nch_u // 2
    mesh = plsc.VectorSubcoreMesh(core_axis_name="c", subcore_axis_name="s")

    def body(head_hbm, a_hbm, msc_hbm,
             e_hbm, dsc_hbm, dpart_hbm,
             m_t, d_t,
             h0, a0, e0, h1, a1, e1,
             mstg_b, acc_b, stg_b,
             sl0, sl1, ss0, ss1):
        c = lax.axis_index("c")
        s = lax.axis_index("s")
        wid = c * NS + s
        _load_merged(msc_hbm, m_t, mstg_b, jnp.maximum, n_ent)
        _fill(d_t, n_ent, 0.0)
        hb = (h0, h1)
        ab = (a0, a1)
        eb = (e0, e1)
        slin = (sl0, sl1)
        sst = (ss0, ss1)

        def cid_of(j):
            raw = wid + NW * j
            real = raw < ncht
            return jnp.minimum(raw, ncht - 1), real

        def issue_lin(q, j):
            cid, _ = cid_of(j)
            base = cid * CH
            pltpu.async_copy(head_hbm.at[pl.ds(base, CH)], hb[q], slin[q])
            pltpu.async_copy(a_hbm.at[pl.ds(base, CH)], ab[q], slin[q])

        def step(q, j):
            pltpu.make_async_copy(eb[q], e_hbm.at[pl.ds(0, CH)], sst[q]).wait()
            pltpu.make_async_copy(head_hbm.at[pl.ds(0, CH)], hb[q], slin[q]).wait()
            pltpu.make_async_copy(a_hbm.at[pl.ds(0, CH)], ab[q], slin[q]).wait()
            cid, real = cid_of(j)
            realf = real.astype(jnp.float32)
            for k in range(KV):
                sl = pl.ds(k * L, L)
                h = hb[q][sl]
                mv = plsc.load_gather(m_t, [h])
                e = jnp.exp(ab[q][sl] - mv) * realf
                eb[q][sl] = e
                plsc.addupdate_scatter(d_t, [h], e)
            base = jnp.where(real, cid * CH, n_edge)
            pltpu.async_copy(eb[q], e_hbm.at[pl.ds(base, CH)], sst[q])
            issue_lin(q, j + 2)

        for q in (0, 1):
            pltpu.async_copy(eb[q], e_hbm.at[pl.ds(n_edge, CH)], sst[q])
        issue_lin(0, 0)
        issue_lin(1, 1)

        def pair(jj, _):
            step(0, 2 * jj)
            step(1, 2 * jj + 1)
            return 0

        lax.fori_loop(0, np_, pair, 0)
        for q in (0, 1):
            pltpu.make_async_copy(eb[q], e_hbm.at[pl.ds(0, CH)], sst[q]).wait()
            pltpu.make_async_copy(head_hbm.at[pl.ds(0, CH)], hb[q], slin[q]).wait()
            pltpu.make_async_copy(a_hbm.at[pl.ds(0, CH)], ab[q], slin[q]).wait()

        pltpu.sync_copy(d_t, dpart_hbm.at[pl.ds(wid * NPAD, n_ent)])
        plsc.subcore_barrier()
        _merge_slice(dpart_hbm, dsc_hbm, c, s, acc_b, stg_b, jnp.add)

    cb_i = pltpu.VMEM((CH,), jnp.int32)
    cb_f = pltpu.VMEM((CH,), jnp.float32)
    return pl.kernel(
        body,
        out_type=(
            jax.ShapeDtypeStruct((n_edge + CH,), jnp.float32),
            jax.ShapeDtypeStruct((NC * NPAD,), jnp.float32),
            jax.ShapeDtypeStruct((NW * NPAD,), jnp.float32),
        ),
        mesh=mesh,
        compiler_params=_SC_PARAMS,
        scratch_types=[
            pltpu.VMEM((n_ent,), jnp.float32),
            pltpu.VMEM((n_ent,), jnp.float32),
            cb_i, cb_f, cb_f, cb_i, cb_f, cb_f,
            pltpu.VMEM((MCH,), jnp.float32),
            pltpu.VMEM((SLICE,), jnp.float32),
            pltpu.VMEM((SLICE,), jnp.float32),
            pltpu.SemaphoreType.DMA, pltpu.SemaphoreType.DMA,
            pltpu.SemaphoreType.DMA, pltpu.SemaphoreType.DMA,
        ],
    )


def _make_sc3(n_ent, n_edge):
    ncht = n_edge // CH
    nch_u = -(-ncht // NS)
    nch_u += nch_u % 2
    np_ = nch_u // 2
    mesh = plsc.VectorSubcoreMesh(core_axis_name="c", subcore_axis_name="s")

    def body(lo_hbm, hi_hbm, head_hbm, g_hbm, e_hbm,
             out_lo, out_hi,
             h0, g0, e0, rows0, hs0, h1, g1, e1, rows1, hs1,
             z_b, st_b, agg,
             sl0, sl1, sr0, sr1, sc0, sc1):
        c = lax.axis_index("c")
        s = lax.axis_index("s")
        hb = (h0, h1)
        gb = (g0, g1)
        eb = (e0, e1)
        rows = (rows0, rows1)
        hsb = (hs0, hs1)
        slin = (sl0, sl1)
        srow = (sr0, sr1)
        sscat = (sc0, sc1)

        for i in range(ZR):
            for j in range(2):
                z_b[i, pl.ds(j * L, L)] = jnp.zeros((L,), jnp.float32)
        off = s * SLICE

        def zloop(q, _):
            pltpu.sync_copy(z_b, agg.at[pl.ds(off + q * ZR, ZR)])
            return 0

        lax.fori_loop(0, SLICE // ZR, zloop, 0)
        plsc.subcore_barrier()

        def cid_of(j):
            raw = s + NS * j
            real = raw < ncht
            return jnp.minimum(raw, ncht - 1), real

        def issue_lin(q, j):
            cid, _ = cid_of(j)
            base = cid * CH
            pltpu.async_copy(head_hbm.at[pl.ds(base, CH)], hb[q], slin[q])
            pltpu.async_copy(g_hbm.at[pl.ds(base, CH)], gb[q], slin[q])
            pltpu.async_copy(e_hbm.at[pl.ds(base, CH)], eb[q], slin[q])

        def wait_lin(q):
            pltpu.make_async_copy(head_hbm.at[pl.ds(0, CH)], hb[q], slin[q]).wait()
            pltpu.make_async_copy(g_hbm.at[pl.ds(0, CH)], gb[q], slin[q]).wait()
            pltpu.make_async_copy(e_hbm.at[pl.ds(0, CH)], eb[q], slin[q]).wait()

        def issue_gather(q):
            @pl.when(c == 0)
            def _():
                pltpu.async_copy(lo_hbm.at[gb[q]], rows[q], srow[q])

            @pl.when(c == 1)
            def _():
                pltpu.async_copy(hi_hbm.at[gb[q]], rows[q], srow[q])

        def wait_gather(q):
            pltpu.make_async_copy(
                lo_hbm.at[pl.ds(0, CH)], rows[q], srow[q]).wait()

        def drain_scat(q):
            pltpu.make_async_copy(rows[q], agg.at[pl.ds(0, CH)], sscat[q]).wait()

        def main(q, j):
            cid, real = cid_of(j)
            realf = real.astype(jnp.float32)
            for k in range(KV):
                sl = pl.ds(k * L, L)
                hsb[q][sl] = hb[q][sl]
                sv = eb[q][sl] * realf
                for i in range(L):
                    row = k * L + i
                    sp = sv[i]
                    rows[q][row, 0:L] = rows[q][row, 0:L] * sp
                    rows[q][row, L:2 * L] = rows[q][row, L:2 * L] * sp
            pltpu.async_copy(rows[q], agg.at[hsb[q]], sscat[q], add=True)

        issue_lin(0, 0)
        issue_lin(1, 1)

        def pair(jj, _):
            j0 = 2 * jj

            @pl.when(jj > 0)
            def _():
                drain_scat(0)
                drain_scat(1)

            wait_lin(0)
            issue_gather(0)
            wait_lin(1)
            issue_gather(1)
            wait_gather(0)
            main(0, j0)
            issue_lin(0, j0 + 2)
            wait_gather(1)
            main(1, j0 + 1)
            issue_lin(1, j0 + 3)
            return 0

        lax.fori_loop(0, np_, pair, 0)
        for q in (0, 1):
            drain_scat(q)
            wait_lin(q)
        plsc.subcore_barrier()

        def drain(q, _):
            pltpu.sync_copy(agg.at[pl.ds(off + q * ZR, ZR)], st_b)

            @pl.when(c == 0)
            def _():
                pltpu.sync_copy(st_b, out_lo.at[pl.ds(off + q * ZR, ZR)])

            @pl.when(c == 1)
            def _():
                pltpu.sync_copy(st_b, out_hi.at[pl.ds(off + q * ZR, ZR)])

            return 0

        lax.fori_loop(0, SLICE // ZR, drain, 0)

    cb_i = pltpu.VMEM((CH,), jnp.int32)
    cb_f = pltpu.VMEM((CH,), jnp.float32)
    rows_t = pltpu.VMEM((CH, 32), jnp.float32)
    return pl.kernel(
        body,
        out_type=(
            jax.ShapeDtypeStruct((NPAD, 32), jnp.float32),
            jax.ShapeDtypeStruct((NPAD, 32), jnp.float32),
        ),
        mesh=mesh,
        compiler_params=_SC_PARAMS,
        scratch_types=[
            cb_i, cb_i, cb_f, rows_t, cb_i,
            cb_i, cb_i, cb_f, rows_t, cb_i,
            pltpu.VMEM((ZR, 32), jnp.float32),
            pltpu.VMEM((ZR, 32), jnp.float32),
            pltpu.VMEM_SHARED((NPAD, 32), jnp.float32),
            pltpu.SemaphoreType.DMA, pltpu.SemaphoreType.DMA,
            pltpu.SemaphoreType.DMA, pltpu.SemaphoreType.DMA,
            pltpu.SemaphoreType.DMA, pltpu.SemaphoreType.DMA,
        ],
    )


def _make_final_body(n_ent, eb, grid):
    tail = n_ent - (grid - 1) * eb

    def body(im_ref, ent_ref, u_ref, lo_ref, hi_ref, d0_ref, d1_ref,
             eagg_ref, uagg_ref):
        i = pl.program_id(0)
        im = im_ref[...]
        ent = ent_ref[...]

        @pl.when(i == grid - 1)
        def _():
            cols = lax.broadcasted_iota(jnp.int32, im.shape, 1)
            rows = lax.broadcasted_iota(jnp.int32, ent.shape, 0)
            im_ref[...] = jnp.where(cols < tail, im, 0.0)
            ent_ref[...] = jnp.where(rows < tail, ent, 0.0)

        imz = im_ref[...]
        base = lax.dot_general(imz, u_ref[...], (((0,), (0,)), ((), ())),
                               preferred_element_type=jnp.float32)
        d = d0_ref[...] + d1_ref[...]
        dinv = 1.0 / jnp.where(d > 0.0, d, 1.0)
        eagg_ref[:, 0:32] = lo_ref[...] * dinv + base[:, 0:32]
        eagg_ref[:, 32:64] = hi_ref[...] * dinv + base[:, 32:64]

        @pl.when(i == 0)
        def _():
            uagg_ref[...] = jnp.zeros_like(uagg_ref)

        uagg_ref[...] += jnp.dot(imz, ent_ref[...],
                                 preferred_element_type=jnp.float32)

    return body


def _tc_final(interact_mat, entity_emb, user_emb, sc_lo, sc_hi, d2d):
    n_usr, n_ent = interact_mat.shape
    d = entity_emb.shape[1]
    eb = 2048
    grid = (n_ent + eb - 1) // eb
    dblk = NPAD // eb
    return pl.pallas_call(
        _make_final_body(n_ent, eb, grid),
        grid=(grid,),
        in_specs=[
            pl.BlockSpec((n_usr, eb), lambda i: (0, i)),
            pl.BlockSpec((eb, d), lambda i: (i, 0)),
            pl.BlockSpec((n_usr, d), lambda i: (0, 0)),
            pl.BlockSpec((eb, 32), lambda i: (i, 0)),
            pl.BlockSpec((eb, 32), lambda i: (i, 0)),
            pl.BlockSpec((eb, 1), lambda i: (i, 0)),
            pl.BlockSpec((eb, 1), lambda i: (i + dblk, 0)),
        ],
        out_specs=[
            pl.BlockSpec((eb, d), lambda i: (i, 0)),
            pl.BlockSpec((n_usr, d), lambda i: (0, 0)),
        ],
        out_shape=[
            jax.ShapeDtypeStruct((n_ent, d), jnp.float32),
            jax.ShapeDtypeStruct((n_usr, d), jnp.float32),
        ],
        compiler_params=pltpu.CompilerParams(
            dimension_semantics=("arbitrary",)),
    )(interact_mat, entity_emb, user_emb, sc_lo, sc_hi, d2d, d2d)


def kernel(entity_emb, user_emb, edge_index, edge_type, interact_mat, weight):
    n_ent = entity_emb.shape[0]
    n_edge = edge_index.shape[1]
    head = edge_index[0]
    tail = edge_index[1]

    lo, hi = _make_sc0(n_ent)(entity_emb.reshape(-1), weight.reshape(-1))
    (nt2,) = _tc_prep(entity_emb, weight)
    a_e, g_idx, m_sc, _ = _make_sc1(n_ent, n_edge)(
        nt2.reshape(-1), head, tail, edge_type)
    e_e, d_sc, _ = _make_sc2(n_ent, n_edge)(head, a_e, m_sc)
    sc_lo, sc_hi = _make_sc3(n_ent, n_edge)(lo, hi, head, g_idx, e_e)
    entity_agg, user_agg = _tc_final(
        interact_mat, entity_emb, user_emb,
        sc_lo, sc_hi, d_sc.reshape(NC * NPAD, 1))
    return (entity_agg, user_agg)


# speedup vs baseline: 12.7684x; 1.0243x over previous
import jax
import jax.numpy as jnp
from jax import lax
from jax.experimental import pallas as pl
from jax.experimental.pallas import tpu as pltpu
from jax.experimental.pallas import tpu_sc as plsc

NC, NS, L = 2, 16, 16
NW = NC * NS
CH = 128
KV = CH // L
SLICE = 3200
NPAD = SLICE * NS
NVS = SLICE // L
MCH = 2000
ZR = 128

_SC_PARAMS = pltpu.CompilerParams(
    needs_layout_passes=False, use_tc_tiling_on_sc=False)

_GDN = lax.GatherDimensionNumbers(
    offset_dims=(), collapsed_slice_dims=(0,), start_index_map=(0,))


def _vgather(x, idx):
    return lax.gather(x, idx[:, None], _GDN, (1,),
                      mode=lax.GatherScatterMode.PROMISE_IN_BOUNDS)


def _seg_update(m_ref, lane_t, idx, val):
    iota = lax.iota(jnp.int32, L)
    fiota = iota.astype(jnp.float32)
    plsc.store_scatter(lane_t, [idx], fiota, mask=idx >= 0)
    got = plsc.load_gather(lane_t, [idx])
    cur = plsc.load_gather(m_ref, [idx])
    leader = got == fiota
    has_dup = jnp.any(jnp.logical_not(leader))

    @pl.when(jnp.logical_not(has_dup))
    def _():
        plsc.store_scatter(m_ref, [idx], val, mask=val > cur)

    @pl.when(has_dup)
    def _():
        vmax = val
        for d in range(1, L):
            src = (iota + d) & (L - 1)
            oi = _vgather(idx, src)
            ov = _vgather(val, src)
            vmax = jnp.where(oi == idx, jnp.maximum(vmax, ov), vmax)
        plsc.store_scatter(m_ref, [idx], vmax, mask=leader & (vmax > cur))


def _merge_slice(part_hbm, out_ref, core, sid, acc_b, stg_b, combine):
    off = sid * SLICE
    row0 = core * NS * NPAD
    pltpu.sync_copy(part_hbm.at[pl.ds(row0 + off, SLICE)], acc_b)

    def one_row(j, _):
        pltpu.sync_copy(part_hbm.at[pl.ds(row0 + j * NPAD + off, SLICE)], stg_b)

        def one_vreg(q, _):
            sl = pl.ds(q * L, L)
            acc_b[sl] = combine(acc_b[sl], stg_b[sl])
            return 0

        return lax.fori_loop(0, NVS, one_vreg, 0)

    lax.fori_loop(1, NS, one_row, 0)
    pltpu.sync_copy(acc_b, out_ref.at[pl.ds(core * NPAD + off, SLICE)])


def _load_merged(src_ref, dst_ref, stg_b, combine, n):
    pltpu.sync_copy(src_ref.at[pl.ds(0, n)], dst_ref.at[pl.ds(0, n)])

    def one_chunk(p, _):
        pltpu.sync_copy(src_ref.at[pl.ds(NPAD + p * MCH, MCH)], stg_b)

        def one_vreg(q, _):
            sl = pl.ds(p * MCH + q * L, L)
            dst_ref[sl] = combine(dst_ref[sl], stg_b[pl.ds(q * L, L)])
            return 0

        return lax.fori_loop(0, MCH // L, one_vreg, 0)

    lax.fori_loop(0, n // MCH, one_chunk, 0)


def _fill(ref, n, value):
    vec = jnp.full((L,), value, ref.dtype)

    def one(i, _):
        ref[pl.ds(i * L, L)] = vec
        return 0

    lax.fori_loop(0, n // L, one, 0)


def _prep_body(ent_ref, w_ref, nt_ref):
    ent = ent_ref[...]
    w = w_ref[...]
    nt_ref[...] = jnp.dot(
        ent * ent, (w * w).T,
        preferred_element_type=jnp.float32,
        precision=lax.Precision.HIGHEST)


def _tc_prep(entity_emb, weight):
    n_ent, d = entity_emb.shape
    n_rel = weight.shape[0]
    rb = 2000
    grid = n_ent // rb
    return pl.pallas_call(
        _prep_body,
        grid=(grid,),
        in_specs=[
            pl.BlockSpec((rb, d), lambda i: (i, 0)),
            pl.BlockSpec((n_rel, d), lambda i: (0, 0)),
        ],
        out_specs=[
            pl.BlockSpec((rb, n_rel), lambda i: (i, 0)),
        ],
        out_shape=[
            jax.ShapeDtypeStruct((n_ent, n_rel), jnp.float32),
        ],
    )(entity_emb, weight)


def _make_sc0(n_ent):
    EC = 8
    ncht = n_ent // EC
    nch_u = -(-ncht // NW)
    nch_u += nch_u % 2
    np_ = nch_u // 2
    rows_c = EC * 16
    mesh = plsc.VectorSubcoreMesh(core_axis_name="c", subcore_axis_name="s")

    def body(ent_hbm, w_hbm, lo_hbm, hi_hbm,
             w_b, e0, lo0, hi0, e1, lo1, hi1,
             sl0, sl1, ss0, ss1):
        c = lax.axis_index("c")
        s = lax.axis_index("s")
        wid = c * NS + s
        pltpu.sync_copy(w_hbm, w_b)
        ebuf = (e0, e1)
        lob = (lo0, lo1)
        hib = (hi0, hi1)
        slin = (sl0, sl1)
        sst = (ss0, ss1)

        def cid_of(j):
            return jnp.minimum(wid + NW * j, ncht - 1)

        def issue_lin(q, j):
            base = cid_of(j) * (EC * 64)
            pltpu.async_copy(ent_hbm.at[pl.ds(base, EC * 64)], ebuf[q], slin[q])

        def wait_lin(q):
            pltpu.make_async_copy(
                ent_hbm.at[pl.ds(0, EC * 64)], ebuf[q], slin[q]).wait()

        def drain_store(q):
            pltpu.make_async_copy(lob[q], lo_hbm.at[pl.ds(0, rows_c)], sst[q]).wait()
            pltpu.make_async_copy(hib[q], hi_hbm.at[pl.ds(0, rows_c)], sst[q]).wait()

        def step(q, j, jj):
            wait_lin(q)

            @pl.when(jj > 0)
            def _():
                drain_store(q)

            for v in range(EC):
                ev = [ebuf[q][pl.ds(v * 64 + dd * L, L)] for dd in range(4)]
                for r in range(16):
                    row = v * 16 + r
                    for dd in range(4):
                        wv = w_b[pl.ds(r * 64 + dd * L, L)]
                        dst = lob[q] if dd < 2 else hib[q]
                        dst[row, pl.ds((dd % 2) * L, L)] = ev[dd] * wv
            base = cid_of(j) * rows_c
            pltpu.async_copy(lob[q], lo_hbm.at[pl.ds(base, rows_c)], sst[q])
            pltpu.async_copy(hib[q], hi_hbm.at[pl.ds(base, rows_c)], sst[q])
            issue_lin(q, j + 2)

        issue_lin(0, 0)
        issue_lin(1, 1)

        def pair(jj, _):
            step(0, 2 * jj, jj)
            step(1, 2 * jj + 1, jj)
            return 0

        lax.fori_loop(0, np_, pair, 0)
        for q in (0, 1):
            drain_store(q)
            wait_lin(q)

    rows_t = pltpu.VMEM((rows_c, 32), jnp.float32)
    return pl.kernel(
        body,
        out_type=(
            jax.ShapeDtypeStruct((n_ent * 16, 32), jnp.float32),
            jax.ShapeDtypeStruct((n_ent * 16, 32), jnp.float32),
        ),
        mesh=mesh,
        compiler_params=_SC_PARAMS,
        scratch_types=[
            pltpu.VMEM((1024,), jnp.float32),
            pltpu.VMEM((EC * 64,), jnp.float32), rows_t, rows_t,
            pltpu.VMEM((EC * 64,), jnp.float32), rows_t, rows_t,
            pltpu.SemaphoreType.DMA, pltpu.SemaphoreType.DMA,
            pltpu.SemaphoreType.DMA, pltpu.SemaphoreType.DMA,
        ],
    )


def _make_sc1(n_ent, n_edge):
    ncht = n_edge // CH
    nch_u = -(-ncht // NW)
    nch_u += nch_u % 2
    np_ = nch_u // 2
    mesh = plsc.VectorSubcoreMesh(core_axis_name="c", subcore_axis_name="s")

    def body(nt_hbm, ei_hbm, et_hbm,
             a_hbm, g_hbm, msc_hbm, mpart_hbm,
             m_t, lane_t,
             h0, t0, r0, gh0, gt0, nh0, ntl0, a0,
             h1, t1, r1, gh1, gt1, nh1, ntl1, a1,
             acc_b, stg_b,
             sl0, sl1, sg0, sg1, ss0, ss1):
        c = lax.axis_index("c")
        s = lax.axis_index("s")
        wid = c * NS + s
        _fill(m_t, n_ent, -1.0)
        hb = (h0, h1)
        tb = (t0, t1)
        rb = (r0, r1)
        ghb = (gh0, gh1)
        gtb = (gt0, gt1)
        nhb = (nh0, nh1)
        ntlb = (ntl0, ntl1)
        ab = (a0, a1)
        slin = (sl0, sl1)
        sgat = (sg0, sg1)
        sst = (ss0, ss1)

        def cid_of(j):
            raw = wid + NW * j
            real = raw < ncht
            return jnp.minimum(raw, ncht - 1), real

        def issue_lin(q, j):
            cid, _ = cid_of(j)
            base = cid * CH
            pltpu.async_copy(ei_hbm.at[0, pl.ds(base, CH)], hb[q], slin[q])
            pltpu.async_copy(ei_hbm.at[1, pl.ds(base, CH)], tb[q], slin[q])
            pltpu.async_copy(et_hbm.at[pl.ds(base, CH)], rb[q], slin[q])

        def wait_lin(q):
            pltpu.make_async_copy(ei_hbm.at[0, pl.ds(0, CH)], hb[q], slin[q]).wait()
            pltpu.make_async_copy(ei_hbm.at[1, pl.ds(0, CH)], tb[q], slin[q]).wait()
            pltpu.make_async_copy(et_hbm.at[pl.ds(0, CH)], rb[q], slin[q]).wait()

        def drain_store(q):
            pltpu.make_async_copy(ab[q], a_hbm.at[pl.ds(0, CH)], sst[q]).wait()
            pltpu.make_async_copy(gtb[q], g_hbm.at[pl.ds(0, CH)], sst[q]).wait()

        def idx_and_gather(q):
            for k in range(KV):
                sl = pl.ds(k * L, L)
                ridx = (rb[q][sl] - 1) & 15
                ghb[q][sl] = hb[q][sl] * 16 + ridx
                gtb[q][sl] = tb[q][sl] * 16 + ridx
            pltpu.async_copy(nt_hbm.at[ghb[q]], nhb[q], sgat[q])
            pltpu.async_copy(nt_hbm.at[gtb[q]], ntlb[q], sgat[q])

        def wait_gather(q):
            pltpu.make_async_copy(nt_hbm.at[pl.ds(0, CH)], nhb[q], sgat[q]).wait()
            pltpu.make_async_copy(nt_hbm.at[pl.ds(0, CH)], ntlb[q], sgat[q]).wait()

        def main(q, j):
            cid, real = cid_of(j)
            realf = real.astype(jnp.float32)
            for k in range(KV):
                sl = pl.ds(k * L, L)
                a = nhb[q][sl] * ntlb[q][sl] * realf - (1.0 - realf)
                ab[q][sl] = a
                _seg_update(m_t, lane_t, hb[q][sl], a)
            base = jnp.where(real, cid * CH, n_edge)
            pltpu.async_copy(ab[q], a_hbm.at[pl.ds(base, CH)], sst[q])
            pltpu.async_copy(gtb[q], g_hbm.at[pl.ds(base, CH)], sst[q])

        for q in (0, 1):
            pltpu.async_copy(ab[q], a_hbm.at[pl.ds(n_edge, CH)], sst[q])
            pltpu.async_copy(gtb[q], g_hbm.at[pl.ds(n_edge, CH)], sst[q])
        issue_lin(0, 0)
        issue_lin(1, 1)

        def pair(jj, _):
            j0 = 2 * jj
            drain_store(0)
            wait_lin(0)
            idx_and_gather(0)
            drain_store(1)
            wait_lin(1)
            idx_and_gather(1)
            wait_gather(0)
            main(0, j0)
            issue_lin(0, j0 + 2)
            wait_gather(1)
            main(1, j0 + 3 - 2)
            issue_lin(1, j0 + 3)
            return 0

        lax.fori_loop(0, np_, pair, 0)
        for q in (0, 1):
            drain_store(q)
            wait_lin(q)

        pltpu.sync_copy(m_t, mpart_hbm.at[pl.ds(wid * NPAD, n_ent)])
        plsc.subcore_barrier()
        _merge_slice(mpart_hbm, msc_hbm, c, s, acc_b, stg_b, jnp.maximum)

    cb_i = pltpu.VMEM((CH,), jnp.int32)
    cb_f = pltpu.VMEM((CH,), jnp.float32)
    return pl.kernel(
        body,
        out_type=(
            jax.ShapeDtypeStruct((n_edge + CH,), jnp.float32),
            jax.ShapeDtypeStruct((n_edge + CH,), jnp.int32),
            jax.ShapeDtypeStruct((NC * NPAD,), jnp.float32),
            jax.ShapeDtypeStruct((NW * NPAD,), jnp.float32),
        ),
        mesh=mesh,
        compiler_params=_SC_PARAMS,
        scratch_types=[
            pltpu.VMEM((n_ent,), jnp.float32),
            pltpu.VMEM((n_ent,), jnp.float32),
            cb_i, cb_i, cb_i, cb_i, cb_i, cb_f, cb_f, cb_f,
            cb_i, cb_i, cb_i, cb_i, cb_i, cb_f, cb_f, cb_f,
            pltpu.VMEM((SLICE,), jnp.float32),
            pltpu.VMEM((SLICE,), jnp.float32),
            pltpu.SemaphoreType.DMA, pltpu.SemaphoreType.DMA,
            pltpu.SemaphoreType.DMA, pltpu.SemaphoreType.DMA,
            pltpu.SemaphoreType.DMA, pltpu.SemaphoreType.DMA,
        ],
    )


def _make_sc2(n_ent, n_edge):
    ncht = n_edge // CH
    nch_u = -(-ncht // NW)
    nch_u += nch_u % 2
    np_ = nch_u // 2
    mesh = plsc.VectorSubcoreMesh(core_axis_name="c", subcore_axis_name="s")

    def body(ei_hbm, a_hbm, msc_hbm,
             e_hbm, dsc_hbm, dpart_hbm,
             m_t, d_t,
             h0, a0, e0, h1, a1, e1,
             mstg_b, acc_b, stg_b,
             sl0, sl1, ss0, ss1):
        c = lax.axis_index("c")
        s = lax.axis_index("s")
        wid = c * NS + s
        _load_merged(msc_hbm, m_t, mstg_b, jnp.maximum, n_ent)
        _fill(d_t, n_ent, 0.0)
        hb = (h0, h1)
        ab = (a0, a1)
        eb = (e0, e1)
        slin = (sl0, sl1)
        sst = (ss0, ss1)

        def cid_of(j):
            raw = wid + NW * j
            real = raw < ncht
            return jnp.minimum(raw, ncht - 1), real

        def issue_lin(q, j):
            cid, _ = cid_of(j)
            base = cid * CH
            pltpu.async_copy(ei_hbm.at[0, pl.ds(base, CH)], hb[q], slin[q])
            pltpu.async_copy(a_hbm.at[pl.ds(base, CH)], ab[q], slin[q])

        def step(q, j):
            pltpu.make_async_copy(eb[q], e_hbm.at[pl.ds(0, CH)], sst[q]).wait()
            pltpu.make_async_copy(ei_hbm.at[0, pl.ds(0, CH)], hb[q], slin[q]).wait()
            pltpu.make_async_copy(a_hbm.at[pl.ds(0, CH)], ab[q], slin[q]).wait()
            cid, real = cid_of(j)
            realf = real.astype(jnp.float32)
            for k in range(KV):
                sl = pl.ds(k * L, L)
                h = hb[q][sl]
                mv = plsc.load_gather(m_t, [h])
                e = jnp.exp(ab[q][sl] - mv) * realf
                eb[q][sl] = e
                plsc.addupdate_scatter(d_t, [h], e)
            base = jnp.where(real, cid * CH, n_edge)
            pltpu.async_copy(eb[q], e_hbm.at[pl.ds(base, CH)], sst[q])
            issue_lin(q, j + 2)

        for q in (0, 1):
            pltpu.async_copy(eb[q], e_hbm.at[pl.ds(n_edge, CH)], sst[q])
        issue_lin(0, 0)
        issue_lin(1, 1)

        def pair(jj, _):
            step(0, 2 * jj)
            step(1, 2 * jj + 1)
            return 0

        lax.fori_loop(0, np_, pair, 0)
        for q in (0, 1):
            pltpu.make_async_copy(eb[q], e_hbm.at[pl.ds(0, CH)], sst[q]).wait()
            pltpu.make_async_copy(ei_hbm.at[0, pl.ds(0, CH)], hb[q], slin[q]).wait()
            pltpu.make_async_copy(a_hbm.at[pl.ds(0, CH)], ab[q], slin[q]).wait()

        pltpu.sync_copy(d_t, dpart_hbm.at[pl.ds(wid * NPAD, n_ent)])
        plsc.subcore_barrier()
        _merge_slice(dpart_hbm, dsc_hbm, c, s, acc_b, stg_b, jnp.add)

    cb_i = pltpu.VMEM((CH,), jnp.int32)
    cb_f = pltpu.VMEM((CH,), jnp.float32)
    return pl.kernel(
        body,
        out_type=(
            jax.ShapeDtypeStruct((n_edge + CH,), jnp.float32),
            jax.ShapeDtypeStruct((NC * NPAD,), jnp.float32),
            jax.ShapeDtypeStruct((NW * NPAD,), jnp.float32),
        ),
        mesh=mesh,
        compiler_params=_SC_PARAMS,
        scratch_types=[
            pltpu.VMEM((n_ent,), jnp.float32),
            pltpu.VMEM((n_ent,), jnp.float32),
            cb_i, cb_f, cb_f, cb_i, cb_f, cb_f,
            pltpu.VMEM((MCH,), jnp.float32),
            pltpu.VMEM((SLICE,), jnp.float32),
            pltpu.VMEM((SLICE,), jnp.float32),
            pltpu.SemaphoreType.DMA, pltpu.SemaphoreType.DMA,
            pltpu.SemaphoreType.DMA, pltpu.SemaphoreType.DMA,
        ],
    )


def _make_sc3(n_ent, n_edge):
    ncht = n_edge // CH
    nch_u = -(-ncht // NS)
    nch_u += nch_u % 2
    np_ = nch_u // 2
    mesh = plsc.VectorSubcoreMesh(core_axis_name="c", subcore_axis_name="s")

    def body(lo_hbm, hi_hbm, ei_hbm, g_hbm, e_hbm,
             out_lo, out_hi,
             h0, g0, e0, rows0, hs0, h1, g1, e1, rows1, hs1,
             z_b, st_b, agg,
             sl0, sl1, sr0, sr1, sc0, sc1):
        c = lax.axis_index("c")
        s = lax.axis_index("s")
        hb = (h0, h1)
        gb = (g0, g1)
        eb = (e0, e1)
        rows = (rows0, rows1)
        hsb = (hs0, hs1)
        slin = (sl0, sl1)
        srow = (sr0, sr1)
        sscat = (sc0, sc1)

        for i in range(ZR):
            for j in range(2):
                z_b[i, pl.ds(j * L, L)] = jnp.zeros((L,), jnp.float32)
        off = s * SLICE

        def zloop(q, _):
            pltpu.sync_copy(z_b, agg.at[pl.ds(off + q * ZR, ZR)])
            return 0

        lax.fori_loop(0, SLICE // ZR, zloop, 0)
        plsc.subcore_barrier()

        def cid_of(j):
            raw = s + NS * j
            real = raw < ncht
            return jnp.minimum(raw, ncht - 1), real

        def issue_lin(q, j):
            cid, _ = cid_of(j)
            base = cid * CH
            pltpu.async_copy(ei_hbm.at[0, pl.ds(base, CH)], hb[q], slin[q])
            pltpu.async_copy(g_hbm.at[pl.ds(base, CH)], gb[q], slin[q])
            pltpu.async_copy(e_hbm.at[pl.ds(base, CH)], eb[q], slin[q])

        def wait_lin(q):
            pltpu.make_async_copy(ei_hbm.at[0, pl.ds(0, CH)], hb[q], slin[q]).wait()
            pltpu.make_async_copy(g_hbm.at[pl.ds(0, CH)], gb[q], slin[q]).wait()
            pltpu.make_async_copy(e_hbm.at[pl.ds(0, CH)], eb[q], slin[q]).wait()

        def issue_gather(q):
            @pl.when(c == 0)
            def _():
                pltpu.async_copy(lo_hbm.at[gb[q]], rows[q], srow[q])

            @pl.when(c == 1)
            def _():
                pltpu.async_copy(hi_hbm.at[gb[q]], rows[q], srow[q])

        def wait_gather(q):
            pltpu.make_async_copy(
                lo_hbm.at[pl.ds(0, CH)], rows[q], srow[q]).wait()

        def drain_scat(q):
            pltpu.make_async_copy(rows[q], agg.at[pl.ds(0, CH)], sscat[q]).wait()

        lane_consts = [jnp.full((L,), i, jnp.int32) for i in range(L)]

        def main(q, j):
            cid, real = cid_of(j)
            realf = real.astype(jnp.float32)
            for k in range(KV):
                sl = pl.ds(k * L, L)
                hsb[q][sl] = hb[q][sl]
                sv = eb[q][sl] * realf
                for i in range(L):
                    row = k * L + i
                    sp = _vgather(sv, lane_consts[i])
                    rows[q][row, 0:L] = rows[q][row, 0:L] * sp
                    rows[q][row, L:2 * L] = rows[q][row, L:2 * L] * sp
            pltpu.async_copy(rows[q], agg.at[hsb[q]], sscat[q], add=True)

        issue_lin(0, 0)
        issue_lin(1, 1)

        def pair(jj, _):
            j0 = 2 * jj

            @pl.when(jj > 0)
            def _():
                drain_scat(0)
                drain_scat(1)

            wait_lin(0)
            issue_gather(0)
            wait_lin(1)
            issue_gather(1)
            wait_gather(0)
            main(0, j0)
            issue_lin(0, j0 + 2)
            wait_gather(1)
            main(1, j0 + 1)
            issue_lin(1, j0 + 3)
            return 0

        lax.fori_loop(0, np_, pair, 0)
        for q in (0, 1):
            drain_scat(q)
            wait_lin(q)
        plsc.subcore_barrier()

        def drain(q, _):
            pltpu.sync_copy(agg.at[pl.ds(off + q * ZR, ZR)], st_b)

            @pl.when(c == 0)
            def _():
                pltpu.sync_copy(st_b, out_lo.at[pl.ds(off + q * ZR, ZR)])

            @pl.when(c == 1)
            def _():
                pltpu.sync_copy(st_b, out_hi.at[pl.ds(off + q * ZR, ZR)])

            return 0

        lax.fori_loop(0, SLICE // ZR, drain, 0)

    cb_i = pltpu.VMEM((CH,), jnp.int32)
    cb_f = pltpu.VMEM((CH,), jnp.float32)
    rows_t = pltpu.VMEM((CH, 32), jnp.float32)
    return pl.kernel(
        body,
        out_type=(
            jax.ShapeDtypeStruct((NPAD, 32), jnp.float32),
            jax.ShapeDtypeStruct((NPAD, 32), jnp.float32),
        ),
        mesh=mesh,
        compiler_params=_SC_PARAMS,
        scratch_types=[
            cb_i, cb_i, cb_f, rows_t, cb_i,
            cb_i, cb_i, cb_f, rows_t, cb_i,
            pltpu.VMEM((ZR, 32), jnp.float32),
            pltpu.VMEM((ZR, 32), jnp.float32),
            pltpu.VMEM_SHARED((NPAD, 32), jnp.float32),
            pltpu.SemaphoreType.DMA, pltpu.SemaphoreType.DMA,
            pltpu.SemaphoreType.DMA, pltpu.SemaphoreType.DMA,
            pltpu.SemaphoreType.DMA, pltpu.SemaphoreType.DMA,
        ],
    )


def _make_final_body(n_ent, eb, grid):
    tail = n_ent - (grid - 1) * eb

    def body(im_ref, ent_ref, u_ref, lo_ref, hi_ref, d0_ref, d1_ref,
             eagg_ref, uagg_ref):
        i = pl.program_id(0)
        im = im_ref[...]
        ent = ent_ref[...]

        @pl.when(i == grid - 1)
        def _():
            cols = lax.broadcasted_iota(jnp.int32, im.shape, 1)
            rows = lax.broadcasted_iota(jnp.int32, ent.shape, 0)
            im_ref[...] = jnp.where(cols < tail, im, 0.0)
            ent_ref[...] = jnp.where(rows < tail, ent, 0.0)

        imz = im_ref[...]
        base = lax.dot_general(imz, u_ref[...], (((0,), (0,)), ((), ())),
                               preferred_element_type=jnp.float32)
        d = d0_ref[...] + d1_ref[...]
        dinv = 1.0 / jnp.where(d > 0.0, d, 1.0)
        eagg_ref[:, 0:32] = lo_ref[...] * dinv + base[:, 0:32]
        eagg_ref[:, 32:64] = hi_ref[...] * dinv + base[:, 32:64]

        @pl.when(i == 0)
        def _():
            uagg_ref[...] = jnp.zeros_like(uagg_ref)

        uagg_ref[...] += jnp.dot(imz, ent_ref[...],
                                 preferred_element_type=jnp.float32)

    return body


def _tc_final(interact_mat, entity_emb, user_emb, sc_lo, sc_hi, d2d):
    n_usr, n_ent = interact_mat.shape
    d = entity_emb.shape[1]
    eb = 2560
    grid = (n_ent + eb - 1) // eb
    dblk = NPAD // eb
    return pl.pallas_call(
        _make_final_body(n_ent, eb, grid),
        grid=(grid,),
        in_specs=[
            pl.BlockSpec((n_usr, eb), lambda i: (0, i)),
            pl.BlockSpec((eb, d), lambda i: (i, 0)),
            pl.BlockSpec((n_usr, d), lambda i: (0, 0)),
            pl.BlockSpec((eb, 32), lambda i: (i, 0)),
            pl.BlockSpec((eb, 32), lambda i: (i, 0)),
            pl.BlockSpec((eb, 1), lambda i: (i, 0)),
            pl.BlockSpec((eb, 1), lambda i: (i + dblk, 0)),
        ],
        out_specs=[
            pl.BlockSpec((eb, d), lambda i: (i, 0)),
            pl.BlockSpec((n_usr, d), lambda i: (0, 0)),
        ],
        out_shape=[
            jax.ShapeDtypeStruct((n_ent, d), jnp.float32),
            jax.ShapeDtypeStruct((n_usr, d), jnp.float32),
        ],
        compiler_params=pltpu.CompilerParams(
            dimension_semantics=("arbitrary",)),
    )(interact_mat, entity_emb, user_emb, sc_lo, sc_hi, d2d, d2d)


def kernel(entity_emb, user_emb, edge_index, edge_type, interact_mat, weight):
    n_ent = entity_emb.shape[0]
    n_edge = edge_index.shape[1]

    lo, hi = _make_sc0(n_ent)(entity_emb.reshape(-1), weight.reshape(-1))
    (nt2,) = _tc_prep(entity_emb, weight)
    a_e, g_idx, m_sc, _ = _make_sc1(n_ent, n_edge)(
        nt2.reshape(-1), edge_index, edge_type)
    e_e, d_sc, _ = _make_sc2(n_ent, n_edge)(edge_index, a_e, m_sc)
    sc_lo, sc_hi = _make_sc3(n_ent, n_edge)(lo, hi, edge_index, g_idx, e_e)
    entity_agg, user_agg = _tc_final(
        interact_mat, entity_emb, user_emb,
        sc_lo, sc_hi, d_sc.reshape(NC * NPAD, 1))
    return (entity_agg, user_agg)


# speedup vs baseline: 13.1530x; 1.0301x over previous
import jax
import jax.numpy as jnp
from jax import lax
from jax.experimental import pallas as pl
from jax.experimental.pallas import tpu as pltpu
from jax.experimental.pallas import tpu_sc as plsc

NC, NS, L = 2, 16, 16
NW = NC * NS
CH = 128
KV = CH // L
SLICE = 3200
NPAD = SLICE * NS
NVS = SLICE // L
MCH = 2000
ZR = 128

_SC_PARAMS = pltpu.CompilerParams(
    needs_layout_passes=False, use_tc_tiling_on_sc=False)

_GDN = lax.GatherDimensionNumbers(
    offset_dims=(), collapsed_slice_dims=(0,), start_index_map=(0,))


def _vgather(x, idx):
    return lax.gather(x, idx[:, None], _GDN, (1,),
                      mode=lax.GatherScatterMode.PROMISE_IN_BOUNDS)


def _seg_update(m_ref, lane_t, idx, val):
    iota = lax.iota(jnp.int32, L)
    fiota = iota.astype(jnp.float32)
    plsc.store_scatter(lane_t, [idx], fiota, mask=idx >= 0)
    got = plsc.load_gather(lane_t, [idx])
    cur = plsc.load_gather(m_ref, [idx])
    leader = got == fiota
    has_dup = jnp.any(jnp.logical_not(leader))

    @pl.when(jnp.logical_not(has_dup))
    def _():
        plsc.store_scatter(m_ref, [idx], val, mask=val > cur)

    @pl.when(has_dup)
    def _():
        vmax = val
        for d in range(1, L):
            src = (iota + d) & (L - 1)
            oi = _vgather(idx, src)
            ov = _vgather(val, src)
            vmax = jnp.where(oi == idx, jnp.maximum(vmax, ov), vmax)
        plsc.store_scatter(m_ref, [idx], vmax, mask=leader & (vmax > cur))


def _merge_slice(part_hbm, out_ref, core, sid, acc_b, stg_b, combine):
    off = sid * SLICE
    row0 = core * NS * NPAD
    pltpu.sync_copy(part_hbm.at[pl.ds(row0 + off, SLICE)], acc_b)

    def one_row(j, _):
        pltpu.sync_copy(part_hbm.at[pl.ds(row0 + j * NPAD + off, SLICE)], stg_b)

        def one_vreg(q, _):
            sl = pl.ds(q * L, L)
            acc_b[sl] = combine(acc_b[sl], stg_b[sl])
            return 0

        return lax.fori_loop(0, NVS, one_vreg, 0)

    lax.fori_loop(1, NS, one_row, 0)
    pltpu.sync_copy(acc_b, out_ref.at[pl.ds(core * NPAD + off, SLICE)])


def _load_merged(src_ref, dst_ref, stg_b, combine, n):
    pltpu.sync_copy(src_ref.at[pl.ds(0, n)], dst_ref.at[pl.ds(0, n)])

    def one_chunk(p, _):
        pltpu.sync_copy(src_ref.at[pl.ds(NPAD + p * MCH, MCH)], stg_b)

        def one_vreg(q, _):
            sl = pl.ds(p * MCH + q * L, L)
            dst_ref[sl] = combine(dst_ref[sl], stg_b[pl.ds(q * L, L)])
            return 0

        return lax.fori_loop(0, MCH // L, one_vreg, 0)

    lax.fori_loop(0, n // MCH, one_chunk, 0)


def _fill(ref, n, value):
    vec = jnp.full((L,), value, ref.dtype)

    def one(i, _):
        ref[pl.ds(i * L, L)] = vec
        return 0

    lax.fori_loop(0, n // L, one, 0)


def _prep_body(ent_ref, w_ref, nt_ref):
    ent = ent_ref[...]
    w = w_ref[...]
    nt_ref[...] = jnp.dot(
        ent * ent, (w * w).T,
        preferred_element_type=jnp.float32,
        precision=lax.Precision.HIGHEST)


def _tc_prep(entity_emb, weight):
    n_ent, d = entity_emb.shape
    n_rel = weight.shape[0]
    rb = 2000
    grid = n_ent // rb
    return pl.pallas_call(
        _prep_body,
        grid=(grid,),
        in_specs=[
            pl.BlockSpec((rb, d), lambda i: (i, 0)),
            pl.BlockSpec((n_rel, d), lambda i: (0, 0)),
        ],
        out_specs=[
            pl.BlockSpec((rb, n_rel), lambda i: (i, 0)),
        ],
        out_shape=[
            jax.ShapeDtypeStruct((n_ent, n_rel), jnp.float32),
        ],
    )(entity_emb, weight)


def _make_sc0(n_ent):
    EC = 8
    ncht = n_ent // EC
    nch_u = -(-ncht // NW)
    nch_u += nch_u % 2
    np_ = nch_u // 2
    rows_c = EC * 16
    mesh = plsc.VectorSubcoreMesh(core_axis_name="c", subcore_axis_name="s")

    def body(ent_hbm, w_hbm, lo_hbm, hi_hbm,
             w_b, e0, lo0, hi0, e1, lo1, hi1,
             sl0, sl1, ss0, ss1):
        c = lax.axis_index("c")
        s = lax.axis_index("s")
        wid = c * NS + s
        pltpu.sync_copy(w_hbm, w_b)
        ebuf = (e0, e1)
        lob = (lo0, lo1)
        hib = (hi0, hi1)
        slin = (sl0, sl1)
        sst = (ss0, ss1)

        def cid_of(j):
            return jnp.minimum(wid + NW * j, ncht - 1)

        def issue_lin(q, j):
            base = cid_of(j) * (EC * 64)
            pltpu.async_copy(ent_hbm.at[pl.ds(base, EC * 64)], ebuf[q], slin[q])

        def wait_lin(q):
            pltpu.make_async_copy(
                ent_hbm.at[pl.ds(0, EC * 64)], ebuf[q], slin[q]).wait()

        def drain_store(q):
            pltpu.make_async_copy(lob[q], lo_hbm.at[pl.ds(0, rows_c)], sst[q]).wait()
            pltpu.make_async_copy(hib[q], hi_hbm.at[pl.ds(0, rows_c)], sst[q]).wait()

        def step(q, j, jj):
            wait_lin(q)

            @pl.when(jj > 0)
            def _():
                drain_store(q)

            for v in range(EC):
                ev = [ebuf[q][pl.ds(v * 64 + dd * 32, 32)] for dd in range(2)]
                for r in range(16):
                    row = v * 16 + r
                    for dd in range(2):
                        wv = w_b[pl.ds(r * 64 + dd * 32, 32)]
                        dst = lob[q] if dd == 0 else hib[q]
                        dst[row, 0:32] = ev[dd] * wv
            base = cid_of(j) * rows_c
            pltpu.async_copy(lob[q], lo_hbm.at[pl.ds(base, rows_c)], sst[q])
            pltpu.async_copy(hib[q], hi_hbm.at[pl.ds(base, rows_c)], sst[q])
            issue_lin(q, j + 2)

        issue_lin(0, 0)
        issue_lin(1, 1)

        def pair(jj, _):
            step(0, 2 * jj, jj)
            step(1, 2 * jj + 1, jj)
            return 0

        lax.fori_loop(0, np_, pair, 0)
        for q in (0, 1):
            drain_store(q)
            wait_lin(q)

    rows_t = pltpu.VMEM((rows_c, 32), jnp.bfloat16)
    return pl.kernel(
        body,
        out_type=(
            jax.ShapeDtypeStruct((n_ent * 16, 32), jnp.bfloat16),
            jax.ShapeDtypeStruct((n_ent * 16, 32), jnp.bfloat16),
        ),
        mesh=mesh,
        compiler_params=_SC_PARAMS,
        scratch_types=[
            pltpu.VMEM((1024,), jnp.bfloat16),
            pltpu.VMEM((EC * 64,), jnp.bfloat16), rows_t, rows_t,
            pltpu.VMEM((EC * 64,), jnp.bfloat16), rows_t, rows_t,
            pltpu.SemaphoreType.DMA, pltpu.SemaphoreType.DMA,
            pltpu.SemaphoreType.DMA, pltpu.SemaphoreType.DMA,
        ],
    )


def _make_sc1(n_ent, n_edge):
    ncht = n_edge // CH
    nch_u = -(-ncht // NW)
    nch_u += nch_u % 2
    np_ = nch_u // 2
    mesh = plsc.VectorSubcoreMesh(core_axis_name="c", subcore_axis_name="s")

    def body(nt_hbm, ei_hbm, et_hbm,
             a_hbm, g_hbm, msc_hbm, mpart_hbm,
             m_t, lane_t,
             h0, t0, r0, gh0, gt0, nh0, ntl0, a0,
             h1, t1, r1, gh1, gt1, nh1, ntl1, a1,
             acc_b, stg_b,
             sl0, sl1, sg0, sg1, ss0, ss1):
        c = lax.axis_index("c")
        s = lax.axis_index("s")
        wid = c * NS + s
        _fill(m_t, n_ent, -1.0)
        hb = (h0, h1)
        tb = (t0, t1)
        rb = (r0, r1)
        ghb = (gh0, gh1)
        gtb = (gt0, gt1)
        nhb = (nh0, nh1)
        ntlb = (ntl0, ntl1)
        ab = (a0, a1)
        slin = (sl0, sl1)
        sgat = (sg0, sg1)
        sst = (ss0, ss1)

        def cid_of(j):
            raw = wid + NW * j
            real = raw < ncht
            return jnp.minimum(raw, ncht - 1), real

        def issue_lin(q, j):
            cid, _ = cid_of(j)
            base = cid * CH
            pltpu.async_copy(ei_hbm.at[0, pl.ds(base, CH)], hb[q], slin[q])
            pltpu.async_copy(ei_hbm.at[1, pl.ds(base, CH)], tb[q], slin[q])
            pltpu.async_copy(et_hbm.at[pl.ds(base, CH)], rb[q], slin[q])

        def wait_lin(q):
            pltpu.make_async_copy(ei_hbm.at[0, pl.ds(0, CH)], hb[q], slin[q]).wait()
            pltpu.make_async_copy(ei_hbm.at[1, pl.ds(0, CH)], tb[q], slin[q]).wait()
            pltpu.make_async_copy(et_hbm.at[pl.ds(0, CH)], rb[q], slin[q]).wait()

        def drain_store(q):
            pltpu.make_async_copy(ab[q], a_hbm.at[pl.ds(0, CH)], sst[q]).wait()
            pltpu.make_async_copy(gtb[q], g_hbm.at[pl.ds(0, CH)], sst[q]).wait()

        def idx_and_gather(q):
            for k in range(KV):
                sl = pl.ds(k * L, L)
                ridx = (rb[q][sl] - 1) & 15
                ghb[q][sl] = hb[q][sl] * 16 + ridx
                gtb[q][sl] = tb[q][sl] * 16 + ridx
            pltpu.async_copy(nt_hbm.at[ghb[q]], nhb[q], sgat[q])
            pltpu.async_copy(nt_hbm.at[gtb[q]], ntlb[q], sgat[q])

        def wait_gather(q):
            pltpu.make_async_copy(nt_hbm.at[pl.ds(0, CH)], nhb[q], sgat[q]).wait()
            pltpu.make_async_copy(nt_hbm.at[pl.ds(0, CH)], ntlb[q], sgat[q]).wait()

        def main(q, j):
            cid, real = cid_of(j)
            realf = real.astype(jnp.float32)
            for k in range(KV):
                sl = pl.ds(k * L, L)
                a = nhb[q][sl] * ntlb[q][sl] * realf - (1.0 - realf)
                ab[q][sl] = a
                _seg_update(m_t, lane_t, hb[q][sl], a)
            base = jnp.where(real, cid * CH, n_edge)
            pltpu.async_copy(ab[q], a_hbm.at[pl.ds(base, CH)], sst[q])
            pltpu.async_copy(gtb[q], g_hbm.at[pl.ds(base, CH)], sst[q])

        for q in (0, 1):
            pltpu.async_copy(ab[q], a_hbm.at[pl.ds(n_edge, CH)], sst[q])
            pltpu.async_copy(gtb[q], g_hbm.at[pl.ds(n_edge, CH)], sst[q])
        issue_lin(0, 0)
        issue_lin(1, 1)

        def pair(jj, _):
            j0 = 2 * jj
            drain_store(0)
            wait_lin(0)
            idx_and_gather(0)
            drain_store(1)
            wait_lin(1)
            idx_and_gather(1)
            wait_gather(0)
            main(0, j0)
            issue_lin(0, j0 + 2)
            wait_gather(1)
            main(1, j0 + 3 - 2)
            issue_lin(1, j0 + 3)
            return 0

        lax.fori_loop(0, np_, pair, 0)
        for q in (0, 1):
            drain_store(q)
            wait_lin(q)

        pltpu.sync_copy(m_t, mpart_hbm.at[pl.ds(wid * NPAD, n_ent)])
        plsc.subcore_barrier()
        _merge_slice(mpart_hbm, msc_hbm, c, s, acc_b, stg_b, jnp.maximum)

    cb_i = pltpu.VMEM((CH,), jnp.int32)
    cb_f = pltpu.VMEM((CH,), jnp.float32)
    return pl.kernel(
        body,
        out_type=(
            jax.ShapeDtypeStruct((n_edge + CH,), jnp.float32),
            jax.ShapeDtypeStruct((n_edge + CH,), jnp.int32),
            jax.ShapeDtypeStruct((NC * NPAD,), jnp.float32),
            jax.ShapeDtypeStruct((NW * NPAD,), jnp.float32),
        ),
        mesh=mesh,
        compiler_params=_SC_PARAMS,
        scratch_types=[
            pltpu.VMEM((n_ent,), jnp.float32),
            pltpu.VMEM((n_ent,), jnp.float32),
            cb_i, cb_i, cb_i, cb_i, cb_i, cb_f, cb_f, cb_f,
            cb_i, cb_i, cb_i, cb_i, cb_i, cb_f, cb_f, cb_f,
            pltpu.VMEM((SLICE,), jnp.float32),
            pltpu.VMEM((SLICE,), jnp.float32),
            pltpu.SemaphoreType.DMA, pltpu.SemaphoreType.DMA,
            pltpu.SemaphoreType.DMA, pltpu.SemaphoreType.DMA,
            pltpu.SemaphoreType.DMA, pltpu.SemaphoreType.DMA,
        ],
    )


def _make_sc2(n_ent, n_edge):
    ncht = n_edge // CH
    nch_u = -(-ncht // NW)
    nch_u += nch_u % 2
    np_ = nch_u // 2
    mesh = plsc.VectorSubcoreMesh(core_axis_name="c", subcore_axis_name="s")

    def body(ei_hbm, a_hbm, msc_hbm,
             e_hbm, dsc_hbm, dpart_hbm,
             m_t, d_t,
             h0, a0, e0, h1, a1, e1,
             mstg_b, acc_b, stg_b,
             sl0, sl1, ss0, ss1):
        c = lax.axis_index("c")
        s = lax.axis_index("s")
        wid = c * NS + s
        _load_merged(msc_hbm, m_t, mstg_b, jnp.maximum, n_ent)
        _fill(d_t, n_ent, 0.0)
        hb = (h0, h1)
        ab = (a0, a1)
        eb = (e0, e1)
        slin = (sl0, sl1)
        sst = (ss0, ss1)

        def cid_of(j):
            raw = wid + NW * j
            real = raw < ncht
            return jnp.minimum(raw, ncht - 1), real

        def issue_lin(q, j):
            cid, _ = cid_of(j)
            base = cid * CH
            pltpu.async_copy(ei_hbm.at[0, pl.ds(base, CH)], hb[q], slin[q])
            pltpu.async_copy(a_hbm.at[pl.ds(base, CH)], ab[q], slin[q])

        def step(q, j):
            pltpu.make_async_copy(eb[q], e_hbm.at[pl.ds(0, CH)], sst[q]).wait()
            pltpu.make_async_copy(ei_hbm.at[0, pl.ds(0, CH)], hb[q], slin[q]).wait()
            pltpu.make_async_copy(a_hbm.at[pl.ds(0, CH)], ab[q], slin[q]).wait()
            cid, real = cid_of(j)
            realf = real.astype(jnp.float32)
            for k in range(KV):
                sl = pl.ds(k * L, L)
                h = hb[q][sl]
                mv = plsc.load_gather(m_t, [h])
                e = jnp.exp(ab[q][sl] - mv) * realf
                eb[q][sl] = e
                plsc.addupdate_scatter(d_t, [h], e)
            base = jnp.where(real, cid * CH, n_edge)
            pltpu.async_copy(eb[q], e_hbm.at[pl.ds(base, CH)], sst[q])
            issue_lin(q, j + 2)

        for q in (0, 1):
            pltpu.async_copy(eb[q], e_hbm.at[pl.ds(n_edge, CH)], sst[q])
        issue_lin(0, 0)
        issue_lin(1, 1)

        def pair(jj, _):
            step(0, 2 * jj)
            step(1, 2 * jj + 1)
            return 0

        lax.fori_loop(0, np_, pair, 0)
        for q in (0, 1):
            pltpu.make_async_copy(eb[q], e_hbm.at[pl.ds(0, CH)], sst[q]).wait()
            pltpu.make_async_copy(ei_hbm.at[0, pl.ds(0, CH)], hb[q], slin[q]).wait()
            pltpu.make_async_copy(a_hbm.at[pl.ds(0, CH)], ab[q], slin[q]).wait()

        pltpu.sync_copy(d_t, dpart_hbm.at[pl.ds(wid * NPAD, n_ent)])
        plsc.subcore_barrier()
        _merge_slice(dpart_hbm, dsc_hbm, c, s, acc_b, stg_b, jnp.add)

    cb_i = pltpu.VMEM((CH,), jnp.int32)
    cb_f = pltpu.VMEM((CH,), jnp.float32)
    return pl.kernel(
        body,
        out_type=(
            jax.ShapeDtypeStruct((n_edge + CH,), jnp.float32),
            jax.ShapeDtypeStruct((NC * NPAD,), jnp.float32),
            jax.ShapeDtypeStruct((NW * NPAD,), jnp.float32),
        ),
        mesh=mesh,
        compiler_params=_SC_PARAMS,
        scratch_types=[
            pltpu.VMEM((n_ent,), jnp.float32),
            pltpu.VMEM((n_ent,), jnp.float32),
            cb_i, cb_f, cb_f, cb_i, cb_f, cb_f,
            pltpu.VMEM((MCH,), jnp.float32),
            pltpu.VMEM((SLICE,), jnp.float32),
            pltpu.VMEM((SLICE,), jnp.float32),
            pltpu.SemaphoreType.DMA, pltpu.SemaphoreType.DMA,
            pltpu.SemaphoreType.DMA, pltpu.SemaphoreType.DMA,
        ],
    )


def _make_sc3(n_ent, n_edge):
    ncht = n_edge // CH
    nch_u = -(-ncht // NS)
    nch_u += nch_u % 2
    np_ = nch_u // 2
    mesh = plsc.VectorSubcoreMesh(core_axis_name="c", subcore_axis_name="s")

    def body(lo_hbm, hi_hbm, ei_hbm, g_hbm, e_hbm,
             out_lo, out_hi,
             h0, g0, e0, rows0, hs0, h1, g1, e1, rows1, hs1,
             z_b, st_b, agg,
             sl0, sl1, sr0, sr1, sc0, sc1):
        c = lax.axis_index("c")
        s = lax.axis_index("s")
        hb = (h0, h1)
        gb = (g0, g1)
        eb = (e0, e1)
        rows = (rows0, rows1)
        hsb = (hs0, hs1)
        slin = (sl0, sl1)
        srow = (sr0, sr1)
        sscat = (sc0, sc1)

        for i in range(ZR):
            z_b[i, 0:32] = jnp.zeros((32,), jnp.bfloat16)
        off = s * SLICE

        def zloop(q, _):
            pltpu.sync_copy(z_b, agg.at[pl.ds(off + q * ZR, ZR)])
            return 0

        lax.fori_loop(0, SLICE // ZR, zloop, 0)
        plsc.subcore_barrier()

        def cid_of(j):
            raw = s + NS * j
            real = raw < ncht
            return jnp.minimum(raw, ncht - 1), real

        def issue_lin(q, j):
            cid, _ = cid_of(j)
            base = cid * CH
            pltpu.async_copy(ei_hbm.at[0, pl.ds(base, CH)], hb[q], slin[q])
            pltpu.async_copy(g_hbm.at[pl.ds(base, CH)], gb[q], slin[q])
            pltpu.async_copy(e_hbm.at[pl.ds(base, CH)], eb[q], slin[q])

        def wait_lin(q):
            pltpu.make_async_copy(ei_hbm.at[0, pl.ds(0, CH)], hb[q], slin[q]).wait()
            pltpu.make_async_copy(g_hbm.at[pl.ds(0, CH)], gb[q], slin[q]).wait()
            pltpu.make_async_copy(e_hbm.at[pl.ds(0, CH)], eb[q], slin[q]).wait()

        def issue_gather(q):
            @pl.when(c == 0)
            def _():
                pltpu.async_copy(lo_hbm.at[gb[q]], rows[q], srow[q])

            @pl.when(c == 1)
            def _():
                pltpu.async_copy(hi_hbm.at[gb[q]], rows[q], srow[q])

        def wait_gather(q):
            pltpu.make_async_copy(
                lo_hbm.at[pl.ds(0, CH)], rows[q], srow[q]).wait()

        def drain_scat(q):
            pltpu.make_async_copy(rows[q], agg.at[pl.ds(0, CH)], sscat[q]).wait()

        lane_consts = [jnp.full((L,), i, jnp.int32) for i in range(L)]

        def main(q, j):
            cid, real = cid_of(j)
            realf = real.astype(jnp.float32)
            for k in range(KV):
                sl = pl.ds(k * L, L)
                hsb[q][sl] = hb[q][sl]
                sv = eb[q][sl] * realf
                for i in range(L):
                    row = k * L + i
                    spl = _vgather(sv, lane_consts[i])
                    sp = plsc.pack(spl, spl, format=plsc.PackFormat.INTERLEAVED)
                    rows[q][row, 0:2 * L] = rows[q][row, 0:2 * L] * sp
            pltpu.async_copy(rows[q], agg.at[hsb[q]], sscat[q], add=True)

        issue_lin(0, 0)
        issue_lin(1, 1)

        def pair(jj, _):
            j0 = 2 * jj

            @pl.when(jj > 0)
            def _():
                drain_scat(0)
                drain_scat(1)

            wait_lin(0)
            issue_gather(0)
            wait_lin(1)
            issue_gather(1)
            wait_gather(0)
            main(0, j0)
            issue_lin(0, j0 + 2)
            wait_gather(1)
            main(1, j0 + 1)
            issue_lin(1, j0 + 3)
            return 0

        lax.fori_loop(0, np_, pair, 0)
        for q in (0, 1):
            drain_scat(q)
            wait_lin(q)
        plsc.subcore_barrier()

        def drain(q, _):
            pltpu.sync_copy(agg.at[pl.ds(off + q * ZR, ZR)], st_b)

            @pl.when(c == 0)
            def _():
                pltpu.sync_copy(st_b, out_lo.at[pl.ds(off + q * ZR, ZR)])

            @pl.when(c == 1)
            def _():
                pltpu.sync_copy(st_b, out_hi.at[pl.ds(off + q * ZR, ZR)])

            return 0

        lax.fori_loop(0, SLICE // ZR, drain, 0)

    cb_i = pltpu.VMEM((CH,), jnp.int32)
    cb_f = pltpu.VMEM((CH,), jnp.float32)
    rows_t = pltpu.VMEM((CH, 32), jnp.bfloat16)
    return pl.kernel(
        body,
        out_type=(
            jax.ShapeDtypeStruct((NPAD, 32), jnp.bfloat16),
            jax.ShapeDtypeStruct((NPAD, 32), jnp.bfloat16),
        ),
        mesh=mesh,
        compiler_params=_SC_PARAMS,
        scratch_types=[
            cb_i, cb_i, cb_f, rows_t, cb_i,
            cb_i, cb_i, cb_f, rows_t, cb_i,
            pltpu.VMEM((ZR, 32), jnp.bfloat16),
            pltpu.VMEM((ZR, 32), jnp.bfloat16),
            pltpu.VMEM_SHARED((NPAD, 32), jnp.bfloat16),
            pltpu.SemaphoreType.DMA, pltpu.SemaphoreType.DMA,
            pltpu.SemaphoreType.DMA, pltpu.SemaphoreType.DMA,
            pltpu.SemaphoreType.DMA, pltpu.SemaphoreType.DMA,
        ],
    )


def _make_final_body(n_ent, eb, grid):
    tail = n_ent - (grid - 1) * eb

    def body(im_ref, ent_ref, u_ref, lo_ref, hi_ref, d0_ref, d1_ref,
             eagg_ref, uagg_ref):
        i = pl.program_id(0)
        im = im_ref[...]
        ent = ent_ref[...]

        @pl.when(i == grid - 1)
        def _():
            cols = lax.broadcasted_iota(jnp.int32, im.shape, 1)
            rows = lax.broadcasted_iota(jnp.int32, ent.shape, 0)
            im_ref[...] = jnp.where(cols < tail, im, 0.0)
            ent_ref[...] = jnp.where(rows < tail, ent, 0.0)

        imz = im_ref[...]
        base = lax.dot_general(imz, u_ref[...], (((0,), (0,)), ((), ())),
                               preferred_element_type=jnp.float32)
        d = d0_ref[...] + d1_ref[...]
        dinv = 1.0 / jnp.where(d > 0.0, d, 1.0)
        eagg_ref[:, 0:32] = lo_ref[...].astype(jnp.float32) * dinv + base[:, 0:32]
        eagg_ref[:, 32:64] = hi_ref[...].astype(jnp.float32) * dinv + base[:, 32:64]

        @pl.when(i == 0)
        def _():
            uagg_ref[...] = jnp.zeros_like(uagg_ref)

        uagg_ref[...] += jnp.dot(imz, ent_ref[...],
                                 preferred_element_type=jnp.float32)

    return body


def _tc_final(interact_mat, entity_emb, user_emb, sc_lo, sc_hi, d2d):
    n_usr, n_ent = interact_mat.shape
    d = entity_emb.shape[1]
    eb = 2560
    grid = (n_ent + eb - 1) // eb
    dblk = NPAD // eb
    return pl.pallas_call(
        _make_final_body(n_ent, eb, grid),
        grid=(grid,),
        in_specs=[
            pl.BlockSpec((n_usr, eb), lambda i: (0, i)),
            pl.BlockSpec((eb, d), lambda i: (i, 0)),
            pl.BlockSpec((n_usr, d), lambda i: (0, 0)),
            pl.BlockSpec((eb, 32), lambda i: (i, 0)),
            pl.BlockSpec((eb, 32), lambda i: (i, 0)),
            pl.BlockSpec((eb, 1), lambda i: (i, 0)),
            pl.BlockSpec((eb, 1), lambda i: (i + dblk, 0)),
        ],
        out_specs=[
            pl.BlockSpec((eb, d), lambda i: (i, 0)),
            pl.BlockSpec((n_usr, d), lambda i: (0, 0)),
        ],
        out_shape=[
            jax.ShapeDtypeStruct((n_ent, d), jnp.float32),
            jax.ShapeDtypeStruct((n_usr, d), jnp.float32),
        ],
        compiler_params=pltpu.CompilerParams(
            dimension_semantics=("arbitrary",)),
    )(interact_mat, entity_emb, user_emb, sc_lo, sc_hi, d2d, d2d)


def kernel(entity_emb, user_emb, edge_index, edge_type, interact_mat, weight):
    n_ent = entity_emb.shape[0]
    n_edge = edge_index.shape[1]

    ent_bf = entity_emb.astype(jnp.bfloat16).reshape(-1)
    w_bf = weight.astype(jnp.bfloat16).reshape(-1)
    lo, hi = _make_sc0(n_ent)(ent_bf, w_bf)
    (nt2,) = _tc_prep(entity_emb, weight)
    a_e, g_idx, m_sc, _ = _make_sc1(n_ent, n_edge)(
        nt2.reshape(-1), edge_index, edge_type)
    e_e, d_sc, _ = _make_sc2(n_ent, n_edge)(edge_index, a_e, m_sc)
    sc_lo, sc_hi = _make_sc3(n_ent, n_edge)(lo, hi, edge_index, g_idx, e_e)
    entity_agg, user_agg = _tc_final(
        interact_mat, entity_emb, user_emb,
        sc_lo, sc_hi, d_sc.reshape(NC * NPAD, 1))
    return (entity_agg, user_agg)


# speedup vs baseline: 14.2433x; 1.0829x over previous
import jax
import jax.numpy as jnp
from jax import lax
from jax.experimental import pallas as pl
from jax.experimental.pallas import tpu as pltpu
from jax.experimental.pallas import tpu_sc as plsc

NC, NS, L = 2, 16, 16
NW = NC * NS
CH = 128
KV = CH // L
SLICE = 3200
NPAD = SLICE * NS
NVS = SLICE // L
MCH = 2000
ZR = 128

_SC_PARAMS = pltpu.CompilerParams(
    needs_layout_passes=False, use_tc_tiling_on_sc=False)

_GDN = lax.GatherDimensionNumbers(
    offset_dims=(), collapsed_slice_dims=(0,), start_index_map=(0,))


def _vgather(x, idx):
    return lax.gather(x, idx[:, None], _GDN, (1,),
                      mode=lax.GatherScatterMode.PROMISE_IN_BOUNDS)


def _seg_update(m_ref, lane_t, idx, val):
    iota = lax.iota(jnp.int32, L)
    fiota = iota.astype(jnp.float32)
    plsc.store_scatter(lane_t, [idx], fiota, mask=idx >= 0)
    got = plsc.load_gather(lane_t, [idx])
    cur = plsc.load_gather(m_ref, [idx])
    leader = got == fiota
    has_dup = jnp.any(jnp.logical_not(leader))

    @pl.when(jnp.logical_not(has_dup))
    def _():
        plsc.store_scatter(m_ref, [idx], val, mask=val > cur)

    @pl.when(has_dup)
    def _():
        vmax = val
        for d in range(1, L):
            src = (iota + d) & (L - 1)
            oi = _vgather(idx, src)
            ov = _vgather(val, src)
            vmax = jnp.where(oi == idx, jnp.maximum(vmax, ov), vmax)
        plsc.store_scatter(m_ref, [idx], vmax, mask=leader & (vmax > cur))


def _merge_slice(part_hbm, out_ref, core, sid, acc_b, stg_b, combine):
    off = sid * SLICE
    row0 = core * NS * NPAD
    pltpu.sync_copy(part_hbm.at[pl.ds(row0 + off, SLICE)], acc_b)

    def one_row(j, _):
        pltpu.sync_copy(part_hbm.at[pl.ds(row0 + j * NPAD + off, SLICE)], stg_b)

        def one_vreg(q, _):
            sl = pl.ds(q * L, L)
            acc_b[sl] = combine(acc_b[sl], stg_b[sl])
            return 0

        return lax.fori_loop(0, NVS, one_vreg, 0)

    lax.fori_loop(1, NS, one_row, 0)
    pltpu.sync_copy(acc_b, out_ref.at[pl.ds(core * NPAD + off, SLICE)])


def _load_merged(src_ref, dst_ref, stg_b, combine, n):
    pltpu.sync_copy(src_ref.at[pl.ds(0, n)], dst_ref.at[pl.ds(0, n)])

    def one_chunk(p, _):
        pltpu.sync_copy(src_ref.at[pl.ds(NPAD + p * MCH, MCH)], stg_b)

        def one_vreg(q, _):
            sl = pl.ds(p * MCH + q * L, L)
            dst_ref[sl] = combine(dst_ref[sl], stg_b[pl.ds(q * L, L)])
            return 0

        return lax.fori_loop(0, MCH // L, one_vreg, 0)

    lax.fori_loop(0, n // MCH, one_chunk, 0)


def _fill(ref, n, value):
    vec = jnp.full((L,), value, ref.dtype)

    def one(i, _):
        ref[pl.ds(i * L, L)] = vec
        return 0

    lax.fori_loop(0, n // L, one, 0)


def _prep_body(ent_ref, w_ref, nt_ref):
    ent = ent_ref[...]
    w = w_ref[...]
    nt_ref[...] = jnp.dot(
        ent * ent, (w * w).T,
        preferred_element_type=jnp.float32,
        precision=lax.Precision.HIGHEST)


def _tc_prep(entity_emb, weight):
    n_ent, d = entity_emb.shape
    n_rel = weight.shape[0]
    rb = 2000
    grid = n_ent // rb
    return pl.pallas_call(
        _prep_body,
        grid=(grid,),
        in_specs=[
            pl.BlockSpec((rb, d), lambda i: (i, 0)),
            pl.BlockSpec((n_rel, d), lambda i: (0, 0)),
        ],
        out_specs=[
            pl.BlockSpec((rb, n_rel), lambda i: (i, 0)),
        ],
        out_shape=[
            jax.ShapeDtypeStruct((n_ent, n_rel), jnp.float32),
        ],
    )(entity_emb, weight)


def _make_sc0(n_ent):
    EC = 8
    ncht = n_ent // EC
    nch_u = -(-ncht // NW)
    nch_u += nch_u % 2
    np_ = nch_u // 2
    rows_c = EC * 16
    mesh = plsc.VectorSubcoreMesh(core_axis_name="c", subcore_axis_name="s")

    def body(ent_hbm, w_hbm, lo_hbm, hi_hbm,
             w_b, e0, lo0, hi0, e1, lo1, hi1,
             sl0, sl1, ss0, ss1):
        c = lax.axis_index("c")
        s = lax.axis_index("s")
        wid = c * NS + s
        pltpu.sync_copy(w_hbm, w_b)
        ebuf = (e0, e1)
        lob = (lo0, lo1)
        hib = (hi0, hi1)
        slin = (sl0, sl1)
        sst = (ss0, ss1)

        def cid_of(j):
            return jnp.minimum(wid + NW * j, ncht - 1)

        def issue_lin(q, j):
            base = cid_of(j) * (EC * 64)
            pltpu.async_copy(ent_hbm.at[pl.ds(base, EC * 64)], ebuf[q], slin[q])

        def wait_lin(q):
            pltpu.make_async_copy(
                ent_hbm.at[pl.ds(0, EC * 64)], ebuf[q], slin[q]).wait()

        def drain_store(q):
            pltpu.make_async_copy(lob[q], lo_hbm.at[pl.ds(0, rows_c)], sst[q]).wait()
            pltpu.make_async_copy(hib[q], hi_hbm.at[pl.ds(0, rows_c)], sst[q]).wait()

        def step(q, j, jj):
            wait_lin(q)

            @pl.when(jj > 0)
            def _():
                drain_store(q)

            for v in range(EC):
                ev = [ebuf[q][pl.ds(v * 64 + dd * 32, 32)] for dd in range(2)]
                for r in range(16):
                    row = v * 16 + r
                    for dd in range(2):
                        wv = w_b[pl.ds(r * 64 + dd * 32, 32)]
                        dst = lob[q] if dd == 0 else hib[q]
                        dst[row, 0:32] = ev[dd] * wv
            base = cid_of(j) * rows_c
            pltpu.async_copy(lob[q], lo_hbm.at[pl.ds(base, rows_c)], sst[q])
            pltpu.async_copy(hib[q], hi_hbm.at[pl.ds(base, rows_c)], sst[q])
            issue_lin(q, j + 2)

        issue_lin(0, 0)
        issue_lin(1, 1)

        def pair(jj, _):
            step(0, 2 * jj, jj)
            step(1, 2 * jj + 1, jj)
            return 0

        lax.fori_loop(0, np_, pair, 0)
        for q in (0, 1):
            drain_store(q)
            wait_lin(q)

    rows_t = pltpu.VMEM((rows_c, 32), jnp.bfloat16)
    return pl.kernel(
        body,
        out_type=(
            jax.ShapeDtypeStruct((n_ent * 16, 32), jnp.bfloat16),
            jax.ShapeDtypeStruct((n_ent * 16, 32), jnp.bfloat16),
        ),
        mesh=mesh,
        compiler_params=_SC_PARAMS,
        scratch_types=[
            pltpu.VMEM((1024,), jnp.bfloat16),
            pltpu.VMEM((EC * 64,), jnp.bfloat16), rows_t, rows_t,
            pltpu.VMEM((EC * 64,), jnp.bfloat16), rows_t, rows_t,
            pltpu.SemaphoreType.DMA, pltpu.SemaphoreType.DMA,
            pltpu.SemaphoreType.DMA, pltpu.SemaphoreType.DMA,
        ],
    )


def _make_sc1(n_ent, n_edge):
    ncht = n_edge // CH
    nch_u = -(-ncht // NW)
    nch_u += nch_u % 2
    np_ = nch_u // 2
    mesh = plsc.VectorSubcoreMesh(core_axis_name="c", subcore_axis_name="s")

    def body(nt_hbm, ei_hbm, et_hbm,
             a_hbm, g_hbm, msc_hbm, mpart_hbm,
             m_t, lane_t,
             h0, t0, r0, gh0, gt0, nh0, ntl0, a0,
             h1, t1, r1, gh1, gt1, nh1, ntl1, a1,
             acc_b, stg_b,
             sl0, sl1, sg0, sg1, ss0, ss1):
        c = lax.axis_index("c")
        s = lax.axis_index("s")
        wid = c * NS + s
        _fill(m_t, n_ent, -1.0)
        hb = (h0, h1)
        tb = (t0, t1)
        rb = (r0, r1)
        ghb = (gh0, gh1)
        gtb = (gt0, gt1)
        nhb = (nh0, nh1)
        ntlb = (ntl0, ntl1)
        ab = (a0, a1)
        slin = (sl0, sl1)
        sgat = (sg0, sg1)
        sst = (ss0, ss1)

        def cid_of(j):
            raw = wid + NW * j
            real = raw < ncht
            return jnp.minimum(raw, ncht - 1), real

        def issue_lin(q, j):
            cid, _ = cid_of(j)
            base = cid * CH
            pltpu.async_copy(ei_hbm.at[0, pl.ds(base, CH)], hb[q], slin[q])
            pltpu.async_copy(ei_hbm.at[1, pl.ds(base, CH)], tb[q], slin[q])
            pltpu.async_copy(et_hbm.at[pl.ds(base, CH)], rb[q], slin[q])

        def wait_lin(q):
            pltpu.make_async_copy(ei_hbm.at[0, pl.ds(0, CH)], hb[q], slin[q]).wait()
            pltpu.make_async_copy(ei_hbm.at[1, pl.ds(0, CH)], tb[q], slin[q]).wait()
            pltpu.make_async_copy(et_hbm.at[pl.ds(0, CH)], rb[q], slin[q]).wait()

        def drain_store(q):
            pltpu.make_async_copy(ab[q], a_hbm.at[pl.ds(0, CH)], sst[q]).wait()
            pltpu.make_async_copy(gtb[q], g_hbm.at[pl.ds(0, CH)], sst[q]).wait()

        def idx_and_gather(q):
            for k in range(KV):
                sl = pl.ds(k * L, L)
                ridx = (rb[q][sl] - 1) & 15
                ghb[q][sl] = hb[q][sl] * 16 + ridx
                gtb[q][sl] = tb[q][sl] * 16 + ridx
            pltpu.async_copy(nt_hbm.at[ghb[q]], nhb[q], sgat[q])
            pltpu.async_copy(nt_hbm.at[gtb[q]], ntlb[q], sgat[q])

        def wait_gather(q):
            pltpu.make_async_copy(nt_hbm.at[pl.ds(0, CH)], nhb[q], sgat[q]).wait()
            pltpu.make_async_copy(nt_hbm.at[pl.ds(0, CH)], ntlb[q], sgat[q]).wait()

        def main(q, j):
            cid, real = cid_of(j)
            realf = real.astype(jnp.float32)
            for k in range(KV):
                sl = pl.ds(k * L, L)
                a = nhb[q][sl] * ntlb[q][sl] * realf - (1.0 - realf)
                ab[q][sl] = a
                _seg_update(m_t, lane_t, hb[q][sl], a)
            base = jnp.where(real, cid * CH, n_edge)
            pltpu.async_copy(ab[q], a_hbm.at[pl.ds(base, CH)], sst[q])
            pltpu.async_copy(gtb[q], g_hbm.at[pl.ds(base, CH)], sst[q])

        for q in (0, 1):
            pltpu.async_copy(ab[q], a_hbm.at[pl.ds(n_edge, CH)], sst[q])
            pltpu.async_copy(gtb[q], g_hbm.at[pl.ds(n_edge, CH)], sst[q])
        issue_lin(0, 0)
        issue_lin(1, 1)

        def pair(jj, _):
            j0 = 2 * jj
            drain_store(0)
            wait_lin(0)
            idx_and_gather(0)
            drain_store(1)
            wait_lin(1)
            idx_and_gather(1)
            wait_gather(0)
            main(0, j0)
            issue_lin(0, j0 + 2)
            wait_gather(1)
            main(1, j0 + 3 - 2)
            issue_lin(1, j0 + 3)
            return 0

        lax.fori_loop(0, np_, pair, 0)
        for q in (0, 1):
            drain_store(q)
            wait_lin(q)

        pltpu.sync_copy(m_t, mpart_hbm.at[pl.ds(wid * NPAD, n_ent)])
        plsc.subcore_barrier()
        _merge_slice(mpart_hbm, msc_hbm, c, s, acc_b, stg_b, jnp.maximum)

    cb_i = pltpu.VMEM((CH,), jnp.int32)
    cb_f = pltpu.VMEM((CH,), jnp.float32)
    return pl.kernel(
        body,
        out_type=(
            jax.ShapeDtypeStruct((n_edge + CH,), jnp.float32),
            jax.ShapeDtypeStruct((n_edge + CH,), jnp.int32),
            jax.ShapeDtypeStruct((NC * NPAD,), jnp.float32),
            jax.ShapeDtypeStruct((NW * NPAD,), jnp.float32),
        ),
        mesh=mesh,
        compiler_params=_SC_PARAMS,
        scratch_types=[
            pltpu.VMEM((n_ent,), jnp.float32),
            pltpu.VMEM((n_ent,), jnp.float32),
            cb_i, cb_i, cb_i, cb_i, cb_i, cb_f, cb_f, cb_f,
            cb_i, cb_i, cb_i, cb_i, cb_i, cb_f, cb_f, cb_f,
            pltpu.VMEM((SLICE,), jnp.float32),
            pltpu.VMEM((SLICE,), jnp.float32),
            pltpu.SemaphoreType.DMA, pltpu.SemaphoreType.DMA,
            pltpu.SemaphoreType.DMA, pltpu.SemaphoreType.DMA,
            pltpu.SemaphoreType.DMA, pltpu.SemaphoreType.DMA,
        ],
    )


def _make_sc2(n_ent, n_edge):
    ncht = n_edge // CH
    nch_u = -(-ncht // NW)
    nch_u += nch_u % 2
    np_ = nch_u // 2
    mesh = plsc.VectorSubcoreMesh(core_axis_name="c", subcore_axis_name="s")

    def body(ei_hbm, a_hbm, msc_hbm,
             e_hbm, dsc_hbm, dpart_hbm,
             m_t, d_t,
             h0, a0, e0, h1, a1, e1,
             mstg_b, acc_b, stg_b,
             sl0, sl1, ss0, ss1):
        c = lax.axis_index("c")
        s = lax.axis_index("s")
        wid = c * NS + s
        _load_merged(msc_hbm, m_t, mstg_b, jnp.maximum, n_ent)
        _fill(d_t, n_ent, 0.0)
        hb = (h0, h1)
        ab = (a0, a1)
        eb = (e0, e1)
        slin = (sl0, sl1)
        sst = (ss0, ss1)

        def cid_of(j):
            raw = wid + NW * j
            real = raw < ncht
            return jnp.minimum(raw, ncht - 1), real

        def issue_lin(q, j):
            cid, _ = cid_of(j)
            base = cid * CH
            pltpu.async_copy(ei_hbm.at[0, pl.ds(base, CH)], hb[q], slin[q])
            pltpu.async_copy(a_hbm.at[pl.ds(base, CH)], ab[q], slin[q])

        def step(q, j):
            pltpu.make_async_copy(eb[q], e_hbm.at[pl.ds(0, CH)], sst[q]).wait()
            pltpu.make_async_copy(ei_hbm.at[0, pl.ds(0, CH)], hb[q], slin[q]).wait()
            pltpu.make_async_copy(a_hbm.at[pl.ds(0, CH)], ab[q], slin[q]).wait()
            cid, real = cid_of(j)
            realf = real.astype(jnp.float32)
            for k in range(KV):
                sl = pl.ds(k * L, L)
                h = hb[q][sl]
                mv = plsc.load_gather(m_t, [h])
                e = jnp.exp(ab[q][sl] - mv) * realf
                eb[q][sl] = e
                plsc.addupdate_scatter(d_t, [h], e)
            base = jnp.where(real, cid * CH, n_edge)
            pltpu.async_copy(eb[q], e_hbm.at[pl.ds(base, CH)], sst[q])
            issue_lin(q, j + 2)

        for q in (0, 1):
            pltpu.async_copy(eb[q], e_hbm.at[pl.ds(n_edge, CH)], sst[q])
        issue_lin(0, 0)
        issue_lin(1, 1)

        def pair(jj, _):
            step(0, 2 * jj)
            step(1, 2 * jj + 1)
            return 0

        lax.fori_loop(0, np_, pair, 0)
        for q in (0, 1):
            pltpu.make_async_copy(eb[q], e_hbm.at[pl.ds(0, CH)], sst[q]).wait()
            pltpu.make_async_copy(ei_hbm.at[0, pl.ds(0, CH)], hb[q], slin[q]).wait()
            pltpu.make_async_copy(a_hbm.at[pl.ds(0, CH)], ab[q], slin[q]).wait()

        pltpu.sync_copy(d_t, dpart_hbm.at[pl.ds(wid * NPAD, n_ent)])
        plsc.subcore_barrier()
        _merge_slice(dpart_hbm, dsc_hbm, c, s, acc_b, stg_b, jnp.add)

    cb_i = pltpu.VMEM((CH,), jnp.int32)
    cb_f = pltpu.VMEM((CH,), jnp.float32)
    return pl.kernel(
        body,
        out_type=(
            jax.ShapeDtypeStruct((n_edge + CH,), jnp.float32),
            jax.ShapeDtypeStruct((NC * NPAD,), jnp.float32),
            jax.ShapeDtypeStruct((NW * NPAD,), jnp.float32),
        ),
        mesh=mesh,
        compiler_params=_SC_PARAMS,
        scratch_types=[
            pltpu.VMEM((n_ent,), jnp.float32),
            pltpu.VMEM((n_ent,), jnp.float32),
            cb_i, cb_f, cb_f, cb_i, cb_f, cb_f,
            pltpu.VMEM((MCH,), jnp.float32),
            pltpu.VMEM((SLICE,), jnp.float32),
            pltpu.VMEM((SLICE,), jnp.float32),
            pltpu.SemaphoreType.DMA, pltpu.SemaphoreType.DMA,
            pltpu.SemaphoreType.DMA, pltpu.SemaphoreType.DMA,
        ],
    )


def _make_sc3(n_ent, n_edge):
    ncht = n_edge // CH
    nch_u = ((-(-ncht // NS)) + 3) // 4 * 4
    np_ = nch_u // 2
    mesh = plsc.VectorSubcoreMesh(core_axis_name="c", subcore_axis_name="s")

    def body(lo_hbm, hi_hbm, ei_hbm, g_hbm, e_hbm,
             out_lo, out_hi,
             h0, g0, e0, rows0, hs0, h1, g1, e1, rows1, hs1,
             h2, g2, e2, rows2, hs2, h3, g3, e3, rows3, hs3,
             z_b, st_b, agg,
             sl0, sl1, sl2, sl3, sr0, sr1, sr2, sr3,
             sc0, sc1, sc2, sc3):
        c = lax.axis_index("c")
        s = lax.axis_index("s")
        hb = (h0, h1, h2, h3)
        gb = (g0, g1, g2, g3)
        eb = (e0, e1, e2, e3)
        rows = (rows0, rows1, rows2, rows3)
        hsb = (hs0, hs1, hs2, hs3)
        slin = (sl0, sl1, sl2, sl3)
        srow = (sr0, sr1, sr2, sr3)
        sscat = (sc0, sc1, sc2, sc3)

        for i in range(ZR):
            z_b[i, 0:32] = jnp.zeros((32,), jnp.bfloat16)
        off = s * SLICE

        def zloop(q, _):
            pltpu.sync_copy(z_b, agg.at[pl.ds(off + q * ZR, ZR)])
            return 0

        lax.fori_loop(0, SLICE // ZR, zloop, 0)
        plsc.subcore_barrier()

        def cid_of(j):
            raw = s + NS * j
            real = raw < ncht
            return jnp.minimum(raw, ncht - 1), real

        def issue_lin(q, j):
            cid, _ = cid_of(j)
            base = cid * CH
            pltpu.async_copy(ei_hbm.at[0, pl.ds(base, CH)], hb[q], slin[q])
            pltpu.async_copy(g_hbm.at[pl.ds(base, CH)], gb[q], slin[q])
            pltpu.async_copy(e_hbm.at[pl.ds(base, CH)], eb[q], slin[q])

        def wait_lin(q):
            pltpu.make_async_copy(ei_hbm.at[0, pl.ds(0, CH)], hb[q], slin[q]).wait()
            pltpu.make_async_copy(g_hbm.at[pl.ds(0, CH)], gb[q], slin[q]).wait()
            pltpu.make_async_copy(e_hbm.at[pl.ds(0, CH)], eb[q], slin[q]).wait()

        def issue_gather(q):
            @pl.when(c == 0)
            def _():
                pltpu.async_copy(lo_hbm.at[gb[q]], rows[q], srow[q])

            @pl.when(c == 1)
            def _():
                pltpu.async_copy(hi_hbm.at[gb[q]], rows[q], srow[q])

        def wait_gather(q):
            pltpu.make_async_copy(
                lo_hbm.at[pl.ds(0, CH)], rows[q], srow[q]).wait()

        def drain_scat(q):
            pltpu.make_async_copy(rows[q], agg.at[pl.ds(0, CH)], sscat[q]).wait()

        lane_consts = [jnp.full((L,), i, jnp.int32) for i in range(L)]

        def main(q, j):
            cid, real = cid_of(j)
            realf = real.astype(jnp.float32)
            for k in range(KV):
                sl = pl.ds(k * L, L)
                hsb[q][sl] = hb[q][sl]
                sv = eb[q][sl] * realf
                for i in range(L):
                    row = k * L + i
                    spl = _vgather(sv, lane_consts[i])
                    sp = plsc.pack(spl, spl, format=plsc.PackFormat.INTERLEAVED)
                    rows[q][row, 0:2 * L] = rows[q][row, 0:2 * L] * sp
            pltpu.async_copy(rows[q], agg.at[hsb[q]], sscat[q], add=True)

        for q in range(4):
            issue_lin(q, q)
        wait_lin(0)
        issue_gather(0)
        wait_lin(1)
        issue_gather(1)

        def quad(jj, _):
            j0 = 4 * jj
            wait_gather(0)
            main(0, j0)
            issue_lin(0, j0 + 4)
            wait_lin(2)

            @pl.when(jj > 0)
            def _():
                drain_scat(2)

            issue_gather(2)
            wait_gather(1)
            main(1, j0 + 1)
            issue_lin(1, j0 + 5)
            wait_lin(3)

            @pl.when(jj > 0)
            def _():
                drain_scat(3)

            issue_gather(3)
            wait_gather(2)
            main(2, j0 + 2)
            issue_lin(2, j0 + 6)
            wait_gather(3)
            main(3, j0 + 3)
            issue_lin(3, j0 + 7)
            wait_lin(0)
            drain_scat(0)
            issue_gather(0)
            wait_lin(1)
            drain_scat(1)
            issue_gather(1)
            return 0

        lax.fori_loop(0, np_ // 2, quad, 0)
        wait_gather(0)
        wait_gather(1)
        drain_scat(2)
        drain_scat(3)
        wait_lin(2)
        wait_lin(3)
        plsc.subcore_barrier()

        def drain(q, _):
            pltpu.sync_copy(agg.at[pl.ds(off + q * ZR, ZR)], st_b)

            @pl.when(c == 0)
            def _():
                pltpu.sync_copy(st_b, out_lo.at[pl.ds(off + q * ZR, ZR)])

            @pl.when(c == 1)
            def _():
                pltpu.sync_copy(st_b, out_hi.at[pl.ds(off + q * ZR, ZR)])

            return 0

        lax.fori_loop(0, SLICE // ZR, drain, 0)

    cb_i = pltpu.VMEM((CH,), jnp.int32)
    cb_f = pltpu.VMEM((CH,), jnp.float32)
    rows_t = pltpu.VMEM((CH, 32), jnp.bfloat16)
    return pl.kernel(
        body,
        out_type=(
            jax.ShapeDtypeStruct((NPAD, 32), jnp.bfloat16),
            jax.ShapeDtypeStruct((NPAD, 32), jnp.bfloat16),
        ),
        mesh=mesh,
        compiler_params=_SC_PARAMS,
        scratch_types=[
            cb_i, cb_i, cb_f, rows_t, cb_i,
            cb_i, cb_i, cb_f, rows_t, cb_i,
            cb_i, cb_i, cb_f, rows_t, cb_i,
            cb_i, cb_i, cb_f, rows_t, cb_i,
            pltpu.VMEM((ZR, 32), jnp.bfloat16),
            pltpu.VMEM((ZR, 32), jnp.bfloat16),
            pltpu.VMEM_SHARED((NPAD, 32), jnp.bfloat16),
        ] + [pltpu.SemaphoreType.DMA] * 12,
    )


def _make_final_body(n_ent, eb, grid):
    tail = n_ent - (grid - 1) * eb

    def body(im_ref, ent_ref, u_ref, lo_ref, hi_ref, d0_ref, d1_ref,
             eagg_ref, uagg_ref):
        i = pl.program_id(0)
        im = im_ref[...]
        ent = ent_ref[...]

        @pl.when(i == grid - 1)
        def _():
            cols = lax.broadcasted_iota(jnp.int32, im.shape, 1)
            rows = lax.broadcasted_iota(jnp.int32, ent.shape, 0)
            im_ref[...] = jnp.where(cols < tail, im, 0.0)
            ent_ref[...] = jnp.where(rows < tail, ent, 0.0)

        imz = im_ref[...]
        base = lax.dot_general(imz, u_ref[...], (((0,), (0,)), ((), ())),
                               preferred_element_type=jnp.float32)
        d = d0_ref[...] + d1_ref[...]
        dinv = 1.0 / jnp.where(d > 0.0, d, 1.0)
        eagg_ref[:, 0:32] = lo_ref[...].astype(jnp.float32) * dinv + base[:, 0:32]
        eagg_ref[:, 32:64] = hi_ref[...].astype(jnp.float32) * dinv + base[:, 32:64]

        @pl.when(i == 0)
        def _():
            uagg_ref[...] = jnp.zeros_like(uagg_ref)

        uagg_ref[...] += jnp.dot(imz, ent_ref[...],
                                 preferred_element_type=jnp.float32)

    return body


def _tc_final(interact_mat, entity_emb, user_emb, sc_lo, sc_hi, d2d):
    n_usr, n_ent = interact_mat.shape
    d = entity_emb.shape[1]
    eb = 2560
    grid = (n_ent + eb - 1) // eb
    dblk = NPAD // eb
    return pl.pallas_call(
        _make_final_body(n_ent, eb, grid),
        grid=(grid,),
        in_specs=[
            pl.BlockSpec((n_usr, eb), lambda i: (0, i)),
            pl.BlockSpec((eb, d), lambda i: (i, 0)),
            pl.BlockSpec((n_usr, d), lambda i: (0, 0)),
            pl.BlockSpec((eb, 32), lambda i: (i, 0)),
            pl.BlockSpec((eb, 32), lambda i: (i, 0)),
            pl.BlockSpec((eb, 1), lambda i: (i, 0)),
            pl.BlockSpec((eb, 1), lambda i: (i + dblk, 0)),
        ],
        out_specs=[
            pl.BlockSpec((eb, d), lambda i: (i, 0)),
            pl.BlockSpec((n_usr, d), lambda i: (0, 0)),
        ],
        out_shape=[
            jax.ShapeDtypeStruct((n_ent, d), jnp.float32),
            jax.ShapeDtypeStruct((n_usr, d), jnp.float32),
        ],
        compiler_params=pltpu.CompilerParams(
            dimension_semantics=("arbitrary",)),
    )(interact_mat, entity_emb, user_emb, sc_lo, sc_hi, d2d, d2d)


def kernel(entity_emb, user_emb, edge_index, edge_type, interact_mat, weight):
    n_ent = entity_emb.shape[0]
    n_edge = edge_index.shape[1]

    ent_bf = entity_emb.astype(jnp.bfloat16).reshape(-1)
    w_bf = weight.astype(jnp.bfloat16).reshape(-1)
    lo, hi = _make_sc0(n_ent)(ent_bf, w_bf)
    (nt2,) = _tc_prep(entity_emb, weight)
    a_e, g_idx, m_sc, _ = _make_sc1(n_ent, n_edge)(
        nt2.reshape(-1), edge_index, edge_type)
    e_e, d_sc, _ = _make_sc2(n_ent, n_edge)(edge_index, a_e, m_sc)
    sc_lo, sc_hi = _make_sc3(n_ent, n_edge)(lo, hi, edge_index, g_idx, e_e)
    entity_agg, user_agg = _tc_final(
        interact_mat, entity_emb, user_emb,
        sc_lo, sc_hi, d_sc.reshape(NC * NPAD, 1))
    return (entity_agg, user_agg)
